# Initial kernel scaffold; baseline (speedup 1.0000x reference)
import jax
import jax.numpy as jnp
from jax.experimental import pallas as pl


def kernel(x, edge_index, W1, att_src1, att_dst1, b1, W2, att_src2, att_dst2, b2):
    raise NotImplementedError("write your pallas kernel here")


# speedup vs baseline: 6.1691x; 6.1691x over previous
import functools

import jax
import jax.numpy as jnp
from jax import lax
from jax.experimental import pallas as pl
from jax.experimental.pallas import tpu as pltpu
from jax.experimental.pallas import tpu_sc as plsc

N = 10000
F = 256
HID = 256
H = 8
CLS = 16
E0 = 160000

NC, NS, L = 2, 16, 16
NW = NC * NS

NPAD = 10240
BLK = 320
NBLK = NPAD // BLK
RPS = NPAD // NS

E = E0 + N
EPT = 5376
EPAD = EPT * NW
CH = 128
NCH_W = EPT // CH
EPT_S = EPAD // NS
NCH_S = EPT_S // CH

_SC_PARAMS = pltpu.CompilerParams(needs_layout_passes=False,
                                  use_tc_tiling_on_sc=False)
_MESH = plsc.VectorSubcoreMesh(core_axis_name="c", subcore_axis_name="s")


def _mm1_body(x_ref, w_ref, asw_ref, adw_ref, h3_ref, asrc_ref, adst_ref):
    hb = jnp.dot(x_ref[...], w_ref[...], preferred_element_type=jnp.float32)
    for s in range(16):
        h3_ref[s, :, :] = hb[:, s * 128:(s + 1) * 128]
    for h in range(H):
        seg = hb[:, h * HID:(h + 1) * HID]
        asrc_ref[:, h:h + 1] = jnp.sum(seg * asw_ref[h:h + 1, :], axis=1,
                                       keepdims=True)
        adst_ref[:, h:h + 1] = jnp.sum(seg * adw_ref[h:h + 1, :], axis=1,
                                       keepdims=True)
    asrc_ref[:, H:] = jnp.zeros((BLK, 16 - H), jnp.float32)
    adst_ref[:, H:] = jnp.zeros((BLK, 16 - H), jnp.float32)


def _mm1(xp, W1, att_src1, att_dst1):
    return pl.pallas_call(
        _mm1_body,
        grid=(NBLK,),
        in_specs=[
            pl.BlockSpec((BLK, F), lambda i: (i, 0)),
            pl.BlockSpec((F, H * HID), lambda i: (0, 0)),
            pl.BlockSpec((H, HID), lambda i: (0, 0)),
            pl.BlockSpec((H, HID), lambda i: (0, 0)),
        ],
        out_specs=[
            pl.BlockSpec((16, BLK, 128), lambda i: (0, i, 0)),
            pl.BlockSpec((BLK, 16), lambda i: (i, 0)),
            pl.BlockSpec((BLK, 16), lambda i: (i, 0)),
        ],
        out_shape=[
            jax.ShapeDtypeStruct((16, NPAD, 128), jnp.float32),
            jax.ShapeDtypeStruct((NPAD, 16), jnp.float32),
            jax.ShapeDtypeStruct((NPAD, 16), jnp.float32),
        ],
    )(xp, W1, att_src1, att_dst1)


@functools.partial(
    pl.kernel,
    out_type=[
        jax.ShapeDtypeStruct((EPAD, 16), jnp.float32),
        jax.ShapeDtypeStruct((NPAD, 16), jnp.float32),
        jax.ShapeDtypeStruct((NPAD, 16), jnp.float32),
    ],
    mesh=_MESH,
    compiler_params=_SC_PARAMS,
    scratch_types=[
        pltpu.VMEM((CH,), jnp.int32),
        pltpu.VMEM((CH,), jnp.int32),
        pltpu.VMEM((CH, 16), jnp.float32),
        pltpu.VMEM((CH, 16), jnp.float32),
        pltpu.VMEM((CH, 16), jnp.float32),
        pltpu.VMEM((RPS, 16), jnp.float32),
        pltpu.VMEM_SHARED((NPAD, 16), jnp.float32),
        pltpu.SemaphoreType.DMA,
        pltpu.SemaphoreType.DMA,
    ],
)
def _edge_softmax(asrc_hbm, adst_hbm, src_hbm, dst_hbm,
                  s_out, d0_out, d1_out,
                  src_v, dst_v, asr, adr, s_blk, zbuf, den_sh, sem1, sem2):
    c = lax.axis_index("c")
    s = lax.axis_index("s")
    wid = c * NS + s

    def zrow(i, carry):
        zbuf[i, :] = jnp.zeros((L,), jnp.float32)
        return carry
    lax.fori_loop(0, RPS, zrow, 0)
    pltpu.sync_copy(zbuf, den_sh.at[pl.ds(s * RPS, RPS)])
    plsc.subcore_barrier()

    def chunk(i, carry):
        base = wid * EPT + i * CH
        pltpu.sync_copy(src_hbm.at[pl.ds(base, CH)], src_v)
        pltpu.sync_copy(dst_hbm.at[pl.ds(base, CH)], dst_v)
        cp1 = pltpu.async_copy(asrc_hbm.at[src_v], asr, sem1)
        cp2 = pltpu.async_copy(adst_hbm.at[dst_v], adr, sem2)
        cp1.wait()
        cp2.wait()

        def row(j, carry2):
            e = asr[j, :] + adr[j, :]
            e = jnp.maximum(e, 0.2 * e)
            s_blk[j, :] = jnp.exp(e)
            return carry2
        lax.fori_loop(0, CH, row, 0)

        pltpu.sync_copy(s_blk, s_out.at[pl.ds(base, CH)])
        pltpu.sync_copy(s_blk, den_sh.at[dst_v], add=True)
        return carry
    lax.fori_loop(0, NCH_W, chunk, 0)
    plsc.subcore_barrier()

    @pl.when(c == 0)
    def _():
        pltpu.sync_copy(den_sh.at[pl.ds(s * RPS, RPS)],
                        d0_out.at[pl.ds(s * RPS, RPS)])

    @pl.when(c == 1)
    def _():
        pltpu.sync_copy(den_sh.at[pl.ds(s * RPS, RPS)],
                        d1_out.at[pl.ds(s * RPS, RPS)])


@functools.partial(
    pl.kernel,
    out_type=jax.ShapeDtypeStruct((16 * NPAD, 128), jnp.float32),
    mesh=_MESH,
    compiler_params=_SC_PARAMS,
    scratch_types=[
        pltpu.VMEM((CH,), jnp.int32),
        pltpu.VMEM((CH,), jnp.int32),
        pltpu.VMEM((CH,), jnp.int32),
        pltpu.VMEM((CH, 16), jnp.float32),
        pltpu.VMEM((CH, 16), jnp.float32),
        pltpu.VMEM((CH, 16), jnp.float32),
        pltpu.VMEM((CH,), jnp.float32),
        pltpu.VMEM((CH, 128), jnp.float32),
        pltpu.VMEM((64, 128), jnp.float32),
        pltpu.VMEM_SHARED((NPAD, 128), jnp.float32),
        pltpu.SemaphoreType.DMA,
        pltpu.SemaphoreType.DMA,
        pltpu.SemaphoreType.DMA,
    ],
)
def _msg1(h3_hbm, src_hbm, dst_hbm, s_hbm, d0_hbm, d1_hbm, out_hbm,
          src_v, dst_v, gidx, s_blk, dr0, dr1, al, rows, zbuf, acc_sh,
          sem1, sem2, sem3):
    c = lax.axis_index("c")
    s = lax.axis_index("s")

    def zrow(i, carry):
        for k in range(128 // L):
            zbuf[i, pl.ds(k * L, L)] = jnp.zeros((L,), jnp.float32)
        return carry
    lax.fori_loop(0, 64, zrow, 0)

    def slab_loop(j, carry):
        slab = c * 8 + j
        head = slab // 2

        def zcp(k, carry2):
            pltpu.sync_copy(zbuf, acc_sh.at[pl.ds(s * RPS + k * 64, 64)])
            return carry2
        lax.fori_loop(0, RPS // 64, zcp, 0)
        plsc.subcore_barrier()

        def chunk(i, carry2):
            base = s * EPT_S + i * CH
            pltpu.sync_copy(src_hbm.at[pl.ds(base, CH)], src_v)
            pltpu.sync_copy(dst_hbm.at[pl.ds(base, CH)], dst_v)
            pltpu.sync_copy(s_hbm.at[pl.ds(base, CH)], s_blk)
            cp1 = pltpu.async_copy(d0_hbm.at[dst_v], dr0, sem1)
            cp2 = pltpu.async_copy(d1_hbm.at[dst_v], dr1, sem2)

            for g in range(CH // L):
                gidx[pl.ds(g * L, L)] = src_v[pl.ds(g * L, L)] + slab * NPAD
            cp3 = pltpu.async_copy(h3_hbm.at[gidx], rows, sem3)
            cp1.wait()
            cp2.wait()

            hv = jnp.full((L,), head, jnp.int32)
            for g in range(CH // L):
                ev = lax.iota(jnp.int32, L) + g * L
                sc = plsc.load_gather(s_blk, [ev, hv])
                dc0 = plsc.load_gather(dr0, [ev, hv])
                dc1 = plsc.load_gather(dr1, [ev, hv])
                al[pl.ds(g * L, L)] = sc / (dc0 + dc1)
            cp3.wait()

            def scale(e, carry3):
                av = plsc.load_gather(al, [jnp.full((L,), e, jnp.int32)])
                for k in range(128 // L):
                    rows[e, pl.ds(k * L, L)] = rows[e, pl.ds(k * L, L)] * av
                return carry3
            lax.fori_loop(0, CH, scale, 0)

            pltpu.sync_copy(rows, acc_sh.at[dst_v], add=True)
            return carry2
        lax.fori_loop(0, NCH_S, chunk, 0)
        plsc.subcore_barrier()

        pltpu.sync_copy(acc_sh.at[pl.ds(s * RPS, RPS)],
                        out_hbm.at[pl.ds(slab * NPAD + s * RPS, RPS)])
        return carry
    lax.fori_loop(0, 8, slab_loop, 0)


def _mm2_body(o1_ref, w2_ref, b1_ref, asw_ref, adw_ref,
              h2_ref, a2s_ref, a2d_ref):
    acc = jnp.zeros((BLK, CLS), jnp.float32)
    for sl in range(16):
        hb = o1_ref[sl] + b1_ref[sl:sl + 1, :]
        hb = jnp.where(hb > 0, hb, jnp.exp(jnp.minimum(hb, 0.0)) - 1.0)
        acc = acc + jnp.dot(hb, w2_ref[sl], preferred_element_type=jnp.float32)
    h2_ref[...] = acc
    a2s = jnp.sum(acc * asw_ref[...], axis=1, keepdims=True)
    a2d = jnp.sum(acc * adw_ref[...], axis=1, keepdims=True)
    a2s_ref[...] = jnp.broadcast_to(a2s, (BLK, 16))
    a2d_ref[...] = jnp.broadcast_to(a2d, (BLK, 16))


def _mm2(out1, W2r, b1r, att_src2, att_dst2):
    return pl.pallas_call(
        _mm2_body,
        grid=(NBLK,),
        in_specs=[
            pl.BlockSpec((16, BLK, 128), lambda i: (0, i, 0)),
            pl.BlockSpec((16, 128, CLS), lambda i: (0, 0, 0)),
            pl.BlockSpec((16, 128), lambda i: (0, 0)),
            pl.BlockSpec((1, CLS), lambda i: (0, 0)),
            pl.BlockSpec((1, CLS), lambda i: (0, 0)),
        ],
        out_specs=[
            pl.BlockSpec((BLK, CLS), lambda i: (i, 0)),
            pl.BlockSpec((BLK, 16), lambda i: (i, 0)),
            pl.BlockSpec((BLK, 16), lambda i: (i, 0)),
        ],
        out_shape=[
            jax.ShapeDtypeStruct((NPAD, CLS), jnp.float32),
            jax.ShapeDtypeStruct((NPAD, 16), jnp.float32),
            jax.ShapeDtypeStruct((NPAD, 16), jnp.float32),
        ],
    )(out1, W2r, b1r, att_src2, att_dst2)


@functools.partial(
    pl.kernel,
    out_type=[
        jax.ShapeDtypeStruct((NPAD, 16), jnp.float32),
        jax.ShapeDtypeStruct((NPAD, 16), jnp.float32),
    ],
    mesh=_MESH,
    compiler_params=_SC_PARAMS,
    scratch_types=[
        pltpu.VMEM((CH,), jnp.int32),
        pltpu.VMEM((CH,), jnp.int32),
        pltpu.VMEM((CH, 16), jnp.float32),
        pltpu.VMEM((CH, 16), jnp.float32),
        pltpu.VMEM((CH, 16), jnp.float32),
        pltpu.VMEM((CH, 16), jnp.float32),
        pltpu.VMEM((RPS, 16), jnp.float32),
        pltpu.VMEM_SHARED((NPAD, 16), jnp.float32),
        pltpu.SemaphoreType.DMA,
        pltpu.SemaphoreType.DMA,
        pltpu.SemaphoreType.DMA,
    ],
)
def _msg2(h2_hbm, src_hbm, dst_hbm, s_hbm, d0_hbm, d1_hbm,
          o0_out, o1_out,
          src_v, dst_v, s_blk, dr0, dr1, rows, zbuf, acc_sh,
          sem1, sem2, sem3):
    c = lax.axis_index("c")
    s = lax.axis_index("s")
    wid = c * NS + s

    def zrow(i, carry):
        zbuf[i, :] = jnp.zeros((L,), jnp.float32)
        return carry
    lax.fori_loop(0, RPS, zrow, 0)
    pltpu.sync_copy(zbuf, acc_sh.at[pl.ds(s * RPS, RPS)])
    plsc.subcore_barrier()

    def chunk(i, carry):
        base = wid * EPT + i * CH
        pltpu.sync_copy(src_hbm.at[pl.ds(base, CH)], src_v)
        pltpu.sync_copy(dst_hbm.at[pl.ds(base, CH)], dst_v)
        pltpu.sync_copy(s_hbm.at[pl.ds(base, CH)], s_blk)
        cp1 = pltpu.async_copy(d0_hbm.at[dst_v], dr0, sem1)
        cp2 = pltpu.async_copy(d1_hbm.at[dst_v], dr1, sem2)
        cp3 = pltpu.async_copy(h2_hbm.at[src_v], rows, sem3)
        cp1.wait()
        cp2.wait()
        cp3.wait()

        def row(e, carry2):
            alpha = s_blk[e, :] / (dr0[e, :] + dr1[e, :])
            rows[e, :] = rows[e, :] * alpha
            return carry2
        lax.fori_loop(0, CH, row, 0)

        pltpu.sync_copy(rows, acc_sh.at[dst_v], add=True)
        return carry
    lax.fori_loop(0, NCH_W, chunk, 0)
    plsc.subcore_barrier()

    @pl.when(c == 0)
    def _():
        pltpu.sync_copy(acc_sh.at[pl.ds(s * RPS, RPS)],
                        o0_out.at[pl.ds(s * RPS, RPS)])

    @pl.when(c == 1)
    def _():
        pltpu.sync_copy(acc_sh.at[pl.ds(s * RPS, RPS)],
                        o1_out.at[pl.ds(s * RPS, RPS)])


def _final_body(p0_ref, p1_ref, b2_ref, o_ref):
    logits = p0_ref[...] + p1_ref[...] + b2_ref[...]
    m = jnp.max(logits, axis=1, keepdims=True)
    ex = jnp.exp(logits - m)
    lse = jnp.log(jnp.sum(ex, axis=1, keepdims=True))
    o_ref[...] = logits - m - lse


def _final(o0, o1, b2r):
    return pl.pallas_call(
        _final_body,
        grid=(NBLK,),
        in_specs=[
            pl.BlockSpec((BLK, CLS), lambda i: (i, 0)),
            pl.BlockSpec((BLK, CLS), lambda i: (i, 0)),
            pl.BlockSpec((1, CLS), lambda i: (0, 0)),
        ],
        out_specs=pl.BlockSpec((BLK, CLS), lambda i: (i, 0)),
        out_shape=jax.ShapeDtypeStruct((NPAD, CLS), jnp.float32),
    )(o0, o1, b2r)


def kernel(x, edge_index, W1, att_src1, att_dst1, b1, W2, att_src2, att_dst2, b2):
    xp = jnp.concatenate(
        [x.astype(jnp.float32), jnp.zeros((NPAD - N, F), jnp.float32)])
    loop = jnp.arange(N, dtype=jnp.int32)
    pad = EPAD - E
    src = jnp.concatenate([edge_index[0].astype(jnp.int32), loop,
                           jnp.zeros((pad,), jnp.int32)])
    dst = jnp.concatenate([edge_index[1].astype(jnp.int32), loop,
                           jnp.full((pad,), N, jnp.int32)])

    h3, asrc1, adst1 = _mm1(xp, W1, att_src1, att_dst1)
    s1, d10, d11 = _edge_softmax(asrc1, adst1, src, dst)
    out1f = _msg1(h3.reshape(16 * NPAD, 128), src, dst, s1, d10, d11)

    h2, a2s, a2d = _mm2(out1f.reshape(16, NPAD, 128),
                        W2.reshape(16, 128, CLS), b1.reshape(16, 128),
                        att_src2, att_dst2)
    s2, d20, d21 = _edge_softmax(a2s, a2d, src, dst)
    o20, o21 = _msg2(h2, src, dst, s2, d20, d21)

    out = _final(o20, o21, b2.reshape(1, CLS))
    return out[:N]


# speedup vs baseline: 7.5477x; 1.2235x over previous
import functools

import jax
import jax.numpy as jnp
from jax import lax
from jax.experimental import pallas as pl
from jax.experimental.pallas import tpu as pltpu
from jax.experimental.pallas import tpu_sc as plsc

N = 10000
F = 256
HID = 256
H = 8
CLS = 16
E0 = 160000

NC, NS, L = 2, 16, 16
NW = NC * NS

NPAD = 10240
BLK = 320
NBLK = NPAD // BLK
RPS = NPAD // NS

E = E0 + N
EPT = 5376
EPAD = EPT * NW
CH = 128
NCH_W = EPT // CH
EPT_S = EPAD // NS
NCH_S = EPT_S // CH

_SC_PARAMS = pltpu.CompilerParams(needs_layout_passes=False,
                                  use_tc_tiling_on_sc=False)
_MESH = plsc.VectorSubcoreMesh(core_axis_name="c", subcore_axis_name="s")


def _mm1_body(x_ref, w_ref, asw_ref, adw_ref, h3_ref, asrc_ref, adst_ref):
    hb = jnp.dot(x_ref[...], w_ref[...], preferred_element_type=jnp.float32)
    for s in range(16):
        h3_ref[s, :, :] = hb[:, s * 128:(s + 1) * 128]
    for h in range(H):
        seg = hb[:, h * HID:(h + 1) * HID]
        asrc_ref[:, h:h + 1] = jnp.sum(seg * asw_ref[h:h + 1, :], axis=1,
                                       keepdims=True)
        adst_ref[:, h:h + 1] = jnp.sum(seg * adw_ref[h:h + 1, :], axis=1,
                                       keepdims=True)
    asrc_ref[:, H:] = jnp.zeros((BLK, 16 - H), jnp.float32)
    adst_ref[:, H:] = jnp.zeros((BLK, 16 - H), jnp.float32)


def _mm1(xp, W1, att_src1, att_dst1):
    return pl.pallas_call(
        _mm1_body,
        grid=(NBLK,),
        in_specs=[
            pl.BlockSpec((BLK, F), lambda i: (i, 0)),
            pl.BlockSpec((F, H * HID), lambda i: (0, 0)),
            pl.BlockSpec((H, HID), lambda i: (0, 0)),
            pl.BlockSpec((H, HID), lambda i: (0, 0)),
        ],
        out_specs=[
            pl.BlockSpec((16, BLK, 128), lambda i: (0, i, 0)),
            pl.BlockSpec((BLK, 16), lambda i: (i, 0)),
            pl.BlockSpec((BLK, 16), lambda i: (i, 0)),
        ],
        out_shape=[
            jax.ShapeDtypeStruct((16, NPAD, 128), jnp.float32),
            jax.ShapeDtypeStruct((NPAD, 16), jnp.float32),
            jax.ShapeDtypeStruct((NPAD, 16), jnp.float32),
        ],
    )(xp, W1, att_src1, att_dst1)


@functools.partial(
    pl.kernel,
    out_type=[
        jax.ShapeDtypeStruct((EPAD, 16), jnp.float32),
        jax.ShapeDtypeStruct((NPAD, 16), jnp.float32),
        jax.ShapeDtypeStruct((NPAD, 16), jnp.float32),
    ],
    mesh=_MESH,
    compiler_params=_SC_PARAMS,
    scratch_types=[
        pltpu.VMEM((CH,), jnp.int32),
        pltpu.VMEM((CH,), jnp.int32),
        pltpu.VMEM((CH, 16), jnp.float32),
        pltpu.VMEM((CH, 16), jnp.float32),
        pltpu.VMEM((CH, 16), jnp.float32),
        pltpu.VMEM((RPS, 16), jnp.float32),
        pltpu.VMEM_SHARED((NPAD, 16), jnp.float32),
        pltpu.SemaphoreType.DMA,
        pltpu.SemaphoreType.DMA,
    ],
)
def _edge_softmax(asrc_hbm, adst_hbm, src_hbm, dst_hbm,
                  s_out, d0_out, d1_out,
                  src_v, dst_v, asr, adr, s_blk, zbuf, den_sh, sem1, sem2):
    c = lax.axis_index("c")
    s = lax.axis_index("s")
    wid = c * NS + s

    def zrow(i, carry):
        zbuf[i, :] = jnp.zeros((L,), jnp.float32)
        return carry
    lax.fori_loop(0, RPS, zrow, 0)
    pltpu.sync_copy(zbuf, den_sh.at[pl.ds(s * RPS, RPS)])
    plsc.subcore_barrier()

    def chunk(i, carry):
        base = wid * EPT + i * CH
        pltpu.sync_copy(src_hbm.at[pl.ds(base, CH)], src_v)
        pltpu.sync_copy(dst_hbm.at[pl.ds(base, CH)], dst_v)
        cp1 = pltpu.async_copy(asrc_hbm.at[src_v], asr, sem1)
        cp2 = pltpu.async_copy(adst_hbm.at[dst_v], adr, sem2)
        cp1.wait()
        cp2.wait()

        def row(j, carry2):
            e = asr[j, :] + adr[j, :]
            e = jnp.maximum(e, 0.2 * e)
            s_blk[j, :] = jnp.exp(e)
            return carry2
        lax.fori_loop(0, CH, row, 0)

        pltpu.sync_copy(s_blk, s_out.at[pl.ds(base, CH)])
        pltpu.sync_copy(s_blk, den_sh.at[dst_v], add=True)
        return carry
    lax.fori_loop(0, NCH_W, chunk, 0)
    plsc.subcore_barrier()

    @pl.when(c == 0)
    def _():
        pltpu.sync_copy(den_sh.at[pl.ds(s * RPS, RPS)],
                        d0_out.at[pl.ds(s * RPS, RPS)])

    @pl.when(c == 1)
    def _():
        pltpu.sync_copy(den_sh.at[pl.ds(s * RPS, RPS)],
                        d1_out.at[pl.ds(s * RPS, RPS)])


@functools.partial(
    pl.kernel,
    out_type=jax.ShapeDtypeStruct((8 * EPAD,), jnp.float32),
    mesh=_MESH,
    compiler_params=_SC_PARAMS,
    scratch_types=[
        pltpu.VMEM((CH,), jnp.int32),
        pltpu.VMEM((CH, 16), jnp.float32),
        pltpu.VMEM((CH, 16), jnp.float32),
        pltpu.VMEM((CH, 16), jnp.float32),
        pltpu.VMEM((8, CH), jnp.float32),
        pltpu.SemaphoreType.DMA,
        pltpu.SemaphoreType.DMA,
    ],
)
def _alpha1(s_hbm, dst_hbm, d0_hbm, d1_hbm, al_out,
            dst_v, s_blk, dr0, dr1, al8, sem1, sem2):
    c = lax.axis_index("c")
    s = lax.axis_index("s")
    wid = c * NS + s

    def chunk(i, carry):
        base = wid * EPT + i * CH
        pltpu.sync_copy(dst_hbm.at[pl.ds(base, CH)], dst_v)
        pltpu.sync_copy(s_hbm.at[pl.ds(base, CH)], s_blk)
        cp1 = pltpu.async_copy(d0_hbm.at[dst_v], dr0, sem1)
        cp2 = pltpu.async_copy(d1_hbm.at[dst_v], dr1, sem2)
        cp1.wait()
        cp2.wait()
        for h in range(H):
            hv = jnp.full((L,), h, jnp.int32)
            for g in range(CH // L):
                ev = lax.iota(jnp.int32, L) + g * L
                sc = plsc.load_gather(s_blk, [ev, hv])
                dc0 = plsc.load_gather(dr0, [ev, hv])
                dc1 = plsc.load_gather(dr1, [ev, hv])
                al8[h, pl.ds(g * L, L)] = sc / (dc0 + dc1)
        for h in range(H):
            pltpu.sync_copy(al8.at[h], al_out.at[pl.ds(h * EPAD + base, CH)])
        return carry
    lax.fori_loop(0, NCH_W, chunk, 0)


@functools.partial(
    pl.kernel,
    out_type=jax.ShapeDtypeStruct((16 * NPAD, 128), jnp.float32),
    mesh=_MESH,
    compiler_params=_SC_PARAMS,
    scratch_types=[
        pltpu.VMEM((CH,), jnp.int32),
        pltpu.VMEM((CH,), jnp.int32),
        pltpu.VMEM((CH,), jnp.int32),
        pltpu.VMEM((CH,), jnp.float32),
        pltpu.VMEM((CH, 128), jnp.float32),
        pltpu.VMEM((64, 128), jnp.float32),
        pltpu.VMEM_SHARED((NPAD, 128), jnp.float32),
        pltpu.SemaphoreType.DMA,
    ],
)
def _msg1(h3_hbm, src_hbm, dst_hbm, al_hbm, out_hbm,
          src_v, dst_v, gidx, al, rows, zbuf, acc_sh, sem3):
    c = lax.axis_index("c")
    s = lax.axis_index("s")

    def zrow(i, carry):
        for k in range(128 // L):
            zbuf[i, pl.ds(k * L, L)] = jnp.zeros((L,), jnp.float32)
        return carry
    lax.fori_loop(0, 64, zrow, 0)

    def slab_loop(j, carry):
        slab = c * 8 + j
        head = slab // 2

        def zcp(k, carry2):
            pltpu.sync_copy(zbuf, acc_sh.at[pl.ds(s * RPS + k * 64, 64)])
            return carry2
        lax.fori_loop(0, RPS // 64, zcp, 0)
        plsc.subcore_barrier()

        def chunk(i, carry2):
            base = s * EPT_S + i * CH
            pltpu.sync_copy(src_hbm.at[pl.ds(base, CH)], src_v)
            for g in range(CH // L):
                gidx[pl.ds(g * L, L)] = src_v[pl.ds(g * L, L)] + slab * NPAD
            cp3 = pltpu.async_copy(h3_hbm.at[gidx], rows, sem3)
            pltpu.sync_copy(dst_hbm.at[pl.ds(base, CH)], dst_v)
            pltpu.sync_copy(al_hbm.at[pl.ds(head * EPAD + base, CH)], al)
            cp3.wait()

            def scale(e, carry3):
                av = plsc.load_gather(al, [jnp.full((L,), e, jnp.int32)])
                for k in range(128 // L):
                    rows[e, pl.ds(k * L, L)] = rows[e, pl.ds(k * L, L)] * av
                return carry3
            lax.fori_loop(0, CH, scale, 0)

            pltpu.sync_copy(rows, acc_sh.at[dst_v], add=True)
            return carry2
        lax.fori_loop(0, NCH_S, chunk, 0)
        plsc.subcore_barrier()

        pltpu.sync_copy(acc_sh.at[pl.ds(s * RPS, RPS)],
                        out_hbm.at[pl.ds(slab * NPAD + s * RPS, RPS)])
        return carry
    lax.fori_loop(0, 8, slab_loop, 0)


def _mm2_body(o1_ref, w2_ref, b1_ref, asw_ref, adw_ref,
              h2_ref, a2s_ref, a2d_ref):
    acc = jnp.zeros((BLK, CLS), jnp.float32)
    for sl in range(16):
        hb = o1_ref[sl] + b1_ref[sl:sl + 1, :]
        hb = jnp.where(hb > 0, hb, jnp.exp(jnp.minimum(hb, 0.0)) - 1.0)
        acc = acc + jnp.dot(hb, w2_ref[sl], preferred_element_type=jnp.float32)
    h2_ref[...] = acc
    a2s = jnp.sum(acc * asw_ref[...], axis=1, keepdims=True)
    a2d = jnp.sum(acc * adw_ref[...], axis=1, keepdims=True)
    a2s_ref[...] = jnp.broadcast_to(a2s, (BLK, 16))
    a2d_ref[...] = jnp.broadcast_to(a2d, (BLK, 16))


def _mm2(out1, W2r, b1r, att_src2, att_dst2):
    return pl.pallas_call(
        _mm2_body,
        grid=(NBLK,),
        in_specs=[
            pl.BlockSpec((16, BLK, 128), lambda i: (0, i, 0)),
            pl.BlockSpec((16, 128, CLS), lambda i: (0, 0, 0)),
            pl.BlockSpec((16, 128), lambda i: (0, 0)),
            pl.BlockSpec((1, CLS), lambda i: (0, 0)),
            pl.BlockSpec((1, CLS), lambda i: (0, 0)),
        ],
        out_specs=[
            pl.BlockSpec((BLK, CLS), lambda i: (i, 0)),
            pl.BlockSpec((BLK, 16), lambda i: (i, 0)),
            pl.BlockSpec((BLK, 16), lambda i: (i, 0)),
        ],
        out_shape=[
            jax.ShapeDtypeStruct((NPAD, CLS), jnp.float32),
            jax.ShapeDtypeStruct((NPAD, 16), jnp.float32),
            jax.ShapeDtypeStruct((NPAD, 16), jnp.float32),
        ],
    )(out1, W2r, b1r, att_src2, att_dst2)


@functools.partial(
    pl.kernel,
    out_type=[
        jax.ShapeDtypeStruct((NPAD, 16), jnp.float32),
        jax.ShapeDtypeStruct((NPAD, 16), jnp.float32),
    ],
    mesh=_MESH,
    compiler_params=_SC_PARAMS,
    scratch_types=[
        pltpu.VMEM((CH,), jnp.int32),
        pltpu.VMEM((CH,), jnp.int32),
        pltpu.VMEM((CH, 16), jnp.float32),
        pltpu.VMEM((CH, 16), jnp.float32),
        pltpu.VMEM((CH, 16), jnp.float32),
        pltpu.VMEM((CH, 16), jnp.float32),
        pltpu.VMEM((RPS, 16), jnp.float32),
        pltpu.VMEM_SHARED((NPAD, 16), jnp.float32),
        pltpu.SemaphoreType.DMA,
        pltpu.SemaphoreType.DMA,
        pltpu.SemaphoreType.DMA,
    ],
)
def _msg2(h2_hbm, src_hbm, dst_hbm, s_hbm, d0_hbm, d1_hbm,
          o0_out, o1_out,
          src_v, dst_v, s_blk, dr0, dr1, rows, zbuf, acc_sh,
          sem1, sem2, sem3):
    c = lax.axis_index("c")
    s = lax.axis_index("s")
    wid = c * NS + s

    def zrow(i, carry):
        zbuf[i, :] = jnp.zeros((L,), jnp.float32)
        return carry
    lax.fori_loop(0, RPS, zrow, 0)
    pltpu.sync_copy(zbuf, acc_sh.at[pl.ds(s * RPS, RPS)])
    plsc.subcore_barrier()

    def chunk(i, carry):
        base = wid * EPT + i * CH
        pltpu.sync_copy(src_hbm.at[pl.ds(base, CH)], src_v)
        pltpu.sync_copy(dst_hbm.at[pl.ds(base, CH)], dst_v)
        pltpu.sync_copy(s_hbm.at[pl.ds(base, CH)], s_blk)
        cp1 = pltpu.async_copy(d0_hbm.at[dst_v], dr0, sem1)
        cp2 = pltpu.async_copy(d1_hbm.at[dst_v], dr1, sem2)
        cp3 = pltpu.async_copy(h2_hbm.at[src_v], rows, sem3)
        cp1.wait()
        cp2.wait()
        cp3.wait()

        def row(e, carry2):
            alpha = s_blk[e, :] / (dr0[e, :] + dr1[e, :])
            rows[e, :] = rows[e, :] * alpha
            return carry2
        lax.fori_loop(0, CH, row, 0)

        pltpu.sync_copy(rows, acc_sh.at[dst_v], add=True)
        return carry
    lax.fori_loop(0, NCH_W, chunk, 0)
    plsc.subcore_barrier()

    @pl.when(c == 0)
    def _():
        pltpu.sync_copy(acc_sh.at[pl.ds(s * RPS, RPS)],
                        o0_out.at[pl.ds(s * RPS, RPS)])

    @pl.when(c == 1)
    def _():
        pltpu.sync_copy(acc_sh.at[pl.ds(s * RPS, RPS)],
                        o1_out.at[pl.ds(s * RPS, RPS)])


def _final_body(p0_ref, p1_ref, b2_ref, o_ref):
    logits = p0_ref[...] + p1_ref[...] + b2_ref[...]
    m = jnp.max(logits, axis=1, keepdims=True)
    ex = jnp.exp(logits - m)
    lse = jnp.log(jnp.sum(ex, axis=1, keepdims=True))
    o_ref[...] = logits - m - lse


def _final(o0, o1, b2r):
    return pl.pallas_call(
        _final_body,
        grid=(NBLK,),
        in_specs=[
            pl.BlockSpec((BLK, CLS), lambda i: (i, 0)),
            pl.BlockSpec((BLK, CLS), lambda i: (i, 0)),
            pl.BlockSpec((1, CLS), lambda i: (0, 0)),
        ],
        out_specs=pl.BlockSpec((BLK, CLS), lambda i: (i, 0)),
        out_shape=jax.ShapeDtypeStruct((NPAD, CLS), jnp.float32),
    )(o0, o1, b2r)


def kernel(x, edge_index, W1, att_src1, att_dst1, b1, W2, att_src2, att_dst2, b2):
    xp = jnp.concatenate(
        [x.astype(jnp.float32), jnp.zeros((NPAD - N, F), jnp.float32)])
    loop = jnp.arange(N, dtype=jnp.int32)
    pad = EPAD - E
    src = jnp.concatenate([edge_index[0].astype(jnp.int32), loop,
                           jnp.zeros((pad,), jnp.int32)])
    dst = jnp.concatenate([edge_index[1].astype(jnp.int32), loop,
                           jnp.full((pad,), N, jnp.int32)])

    h3, asrc1, adst1 = _mm1(xp, W1, att_src1, att_dst1)
    s1, d10, d11 = _edge_softmax(asrc1, adst1, src, dst)
    al1 = _alpha1(s1, dst, d10, d11)
    out1f = _msg1(h3.reshape(16 * NPAD, 128), src, dst, al1)

    h2, a2s, a2d = _mm2(out1f.reshape(16, NPAD, 128),
                        W2.reshape(16, 128, CLS), b1.reshape(16, 128),
                        att_src2, att_dst2)
    s2, d20, d21 = _edge_softmax(a2s, a2d, src, dst)
    o20, o21 = _msg2(h2, src, dst, s2, d20, d21)

    out = _final(o20, o21, b2.reshape(1, CLS))
    return out[:N]


# speedup vs baseline: 8.3871x; 1.1112x over previous
import functools

import jax
import jax.numpy as jnp
from jax import lax
from jax.experimental import pallas as pl
from jax.experimental.pallas import tpu as pltpu
from jax.experimental.pallas import tpu_sc as plsc

N = 10000
F = 256
HID = 256
H = 8
CLS = 16
E0 = 160000

NC, NS, L = 2, 16, 16
NW = NC * NS

NPAD = 10240
BLK = 320
NBLK = NPAD // BLK
RPS = NPAD // NS

E = E0 + N
EPT = 5376
EPAD = EPT * NW
CH = 128
NCH_W = EPT // CH
EPT_S = EPAD // NS
NCH_S = EPT_S // CH

_SC_PARAMS = pltpu.CompilerParams(needs_layout_passes=False,
                                  use_tc_tiling_on_sc=False)
_MESH = plsc.VectorSubcoreMesh(core_axis_name="c", subcore_axis_name="s")


def _mm1_body(x_ref, w_ref, asw_ref, adw_ref, h3_ref, asrc_ref, adst_ref):
    hb = jnp.dot(x_ref[...], w_ref[...], preferred_element_type=jnp.float32)
    for s in range(16):
        h3_ref[s, :, :] = hb[:, s * 128:(s + 1) * 128]
    for h in range(H):
        seg = hb[:, h * HID:(h + 1) * HID]
        asrc_ref[:, h:h + 1] = jnp.sum(seg * asw_ref[h:h + 1, :], axis=1,
                                       keepdims=True)
        adst_ref[:, h:h + 1] = jnp.sum(seg * adw_ref[h:h + 1, :], axis=1,
                                       keepdims=True)
    asrc_ref[:, H:] = jnp.zeros((BLK, 16 - H), jnp.float32)
    adst_ref[:, H:] = jnp.zeros((BLK, 16 - H), jnp.float32)


def _mm1(xp, W1, att_src1, att_dst1):
    return pl.pallas_call(
        _mm1_body,
        grid=(NBLK,),
        in_specs=[
            pl.BlockSpec((BLK, F), lambda i: (i, 0)),
            pl.BlockSpec((F, H * HID), lambda i: (0, 0)),
            pl.BlockSpec((H, HID), lambda i: (0, 0)),
            pl.BlockSpec((H, HID), lambda i: (0, 0)),
        ],
        out_specs=[
            pl.BlockSpec((16, BLK, 128), lambda i: (0, i, 0)),
            pl.BlockSpec((BLK, 16), lambda i: (i, 0)),
            pl.BlockSpec((BLK, 16), lambda i: (i, 0)),
        ],
        out_shape=[
            jax.ShapeDtypeStruct((16, NPAD, 128), jnp.float32),
            jax.ShapeDtypeStruct((NPAD, 16), jnp.float32),
            jax.ShapeDtypeStruct((NPAD, 16), jnp.float32),
        ],
    )(xp, W1, att_src1, att_dst1)


@functools.partial(
    pl.kernel,
    out_type=[
        jax.ShapeDtypeStruct((EPAD, 16), jnp.float32),
        jax.ShapeDtypeStruct((NPAD, 16), jnp.float32),
        jax.ShapeDtypeStruct((NPAD, 16), jnp.float32),
    ],
    mesh=_MESH,
    compiler_params=_SC_PARAMS,
    scratch_types=[
        pltpu.VMEM((CH,), jnp.int32),
        pltpu.VMEM((CH,), jnp.int32),
        pltpu.VMEM((CH, 16), jnp.float32),
        pltpu.VMEM((CH, 16), jnp.float32),
        pltpu.VMEM((CH, 16), jnp.float32),
        pltpu.VMEM((RPS, 16), jnp.float32),
        pltpu.VMEM_SHARED((NPAD, 16), jnp.float32),
        pltpu.SemaphoreType.DMA,
        pltpu.SemaphoreType.DMA,
    ],
)
def _edge_softmax(asrc_hbm, adst_hbm, src_hbm, dst_hbm,
                  s_out, d0_out, d1_out,
                  src_v, dst_v, asr, adr, s_blk, zbuf, den_sh, sem1, sem2):
    c = lax.axis_index("c")
    s = lax.axis_index("s")
    wid = c * NS + s

    def zrow(i, carry):
        zbuf[i, :] = jnp.zeros((L,), jnp.float32)
        return carry
    lax.fori_loop(0, RPS, zrow, 0)
    pltpu.sync_copy(zbuf, den_sh.at[pl.ds(s * RPS, RPS)])
    plsc.subcore_barrier()

    def chunk(i, carry):
        base = wid * EPT + i * CH
        pltpu.sync_copy(src_hbm.at[pl.ds(base, CH)], src_v)
        pltpu.sync_copy(dst_hbm.at[pl.ds(base, CH)], dst_v)
        cp1 = pltpu.async_copy(asrc_hbm.at[src_v], asr, sem1)
        cp2 = pltpu.async_copy(adst_hbm.at[dst_v], adr, sem2)
        cp1.wait()
        cp2.wait()

        def row(j, carry2):
            e = asr[j, :] + adr[j, :]
            e = jnp.maximum(e, 0.2 * e)
            s_blk[j, :] = jnp.exp(e)
            return carry2
        lax.fori_loop(0, CH, row, 0)

        pltpu.sync_copy(s_blk, s_out.at[pl.ds(base, CH)])
        pltpu.sync_copy(s_blk, den_sh.at[dst_v], add=True)
        return carry
    lax.fori_loop(0, NCH_W, chunk, 0)
    plsc.subcore_barrier()

    @pl.when(c == 0)
    def _():
        pltpu.sync_copy(den_sh.at[pl.ds(s * RPS, RPS)],
                        d0_out.at[pl.ds(s * RPS, RPS)])

    @pl.when(c == 1)
    def _():
        pltpu.sync_copy(den_sh.at[pl.ds(s * RPS, RPS)],
                        d1_out.at[pl.ds(s * RPS, RPS)])


@functools.partial(
    pl.kernel,
    out_type=jax.ShapeDtypeStruct((8 * EPAD,), jnp.float32),
    mesh=_MESH,
    compiler_params=_SC_PARAMS,
    scratch_types=[
        pltpu.VMEM((CH,), jnp.int32),
        pltpu.VMEM((CH, 16), jnp.float32),
        pltpu.VMEM((CH, 16), jnp.float32),
        pltpu.VMEM((CH, 16), jnp.float32),
        pltpu.VMEM((8, CH), jnp.float32),
        pltpu.SemaphoreType.DMA,
        pltpu.SemaphoreType.DMA,
    ],
)
def _alpha1(s_hbm, dst_hbm, d0_hbm, d1_hbm, al_out,
            dst_v, s_blk, dr0, dr1, al8, sem1, sem2):
    c = lax.axis_index("c")
    s = lax.axis_index("s")
    wid = c * NS + s

    def chunk(i, carry):
        base = wid * EPT + i * CH
        pltpu.sync_copy(dst_hbm.at[pl.ds(base, CH)], dst_v)
        pltpu.sync_copy(s_hbm.at[pl.ds(base, CH)], s_blk)
        cp1 = pltpu.async_copy(d0_hbm.at[dst_v], dr0, sem1)
        cp2 = pltpu.async_copy(d1_hbm.at[dst_v], dr1, sem2)
        cp1.wait()
        cp2.wait()
        for h in range(H):
            hv = jnp.full((L,), h, jnp.int32)
            for g in range(CH // L):
                ev = lax.iota(jnp.int32, L) + g * L
                sc = plsc.load_gather(s_blk, [ev, hv])
                dc0 = plsc.load_gather(dr0, [ev, hv])
                dc1 = plsc.load_gather(dr1, [ev, hv])
                al8[h, pl.ds(g * L, L)] = sc / (dc0 + dc1)
        for h in range(H):
            pltpu.sync_copy(al8.at[h], al_out.at[pl.ds(h * EPAD + base, CH)])
        return carry
    lax.fori_loop(0, NCH_W, chunk, 0)


@functools.partial(
    pl.kernel,
    out_type=jax.ShapeDtypeStruct((16 * NPAD, 128), jnp.float32),
    mesh=_MESH,
    compiler_params=_SC_PARAMS,
    scratch_types=[
        pltpu.VMEM((CH,), jnp.int32),
        pltpu.VMEM((CH,), jnp.int32),
        pltpu.VMEM((CH,), jnp.int32),
        pltpu.VMEM((CH,), jnp.int32),
        pltpu.VMEM((CH,), jnp.int32),
        pltpu.VMEM((CH,), jnp.int32),
        pltpu.VMEM((CH,), jnp.float32),
        pltpu.VMEM((CH,), jnp.float32),
        pltpu.VMEM((CH, 128), jnp.float32),
        pltpu.VMEM((CH, 128), jnp.float32),
        pltpu.VMEM((64, 128), jnp.float32),
        pltpu.VMEM_SHARED((NPAD, 128), jnp.float32),
        pltpu.SemaphoreType.DMA,
        pltpu.SemaphoreType.DMA,
        pltpu.SemaphoreType.DMA,
        pltpu.SemaphoreType.DMA,
    ],
)
def _msg1(h3_hbm, src_hbm, dst_hbm, al_hbm, out_hbm,
          src_a, src_b, dst_a, dst_b, gidx_a, gidx_b, al_a, al_b,
          rows_a, rows_b, zbuf, acc_sh, sem_ga, sem_gb, sem_sa, sem_sb):
    c = lax.axis_index("c")
    s = lax.axis_index("s")

    def zrow(i, carry):
        for k in range(128 // L):
            zbuf[i, pl.ds(k * L, L)] = jnp.zeros((L,), jnp.float32)
        return carry
    lax.fori_loop(0, 64, zrow, 0)

    def _load_meta(i, slab, head, src_v, dst_v, gidx, al):
        base = s * EPT_S + i * CH
        pltpu.sync_copy(src_hbm.at[pl.ds(base, CH)], src_v)
        for g in range(CH // L):
            gidx[pl.ds(g * L, L)] = src_v[pl.ds(g * L, L)] + slab * NPAD
        pltpu.sync_copy(dst_hbm.at[pl.ds(base, CH)], dst_v)
        pltpu.sync_copy(al_hbm.at[pl.ds(head * EPAD + base, CH)], al)

    def _scale(rows, al):
        def scale(e, carry3):
            e0 = 2 * e
            e1 = 2 * e + 1
            av0 = plsc.load_gather(al, [jnp.full((L,), e0, jnp.int32)])
            av1 = plsc.load_gather(al, [jnp.full((L,), e1, jnp.int32)])
            for k in range(128 // L):
                rows[e0, pl.ds(k * L, L)] = rows[e0, pl.ds(k * L, L)] * av0
                rows[e1, pl.ds(k * L, L)] = rows[e1, pl.ds(k * L, L)] * av1
            return carry3
        lax.fori_loop(0, CH // 2, scale, 0)

    def slab_loop(j, carry):
        slab = c * 8 + j
        head = slab // 2

        def zcp(k, carry2):
            pltpu.sync_copy(zbuf, acc_sh.at[pl.ds(s * RPS + k * 64, 64)])
            return carry2
        lax.fori_loop(0, RPS // 64, zcp, 0)
        plsc.subcore_barrier()

        _load_meta(0, slab, head, src_a, dst_a, gidx_a, al_a)
        cp_a0 = pltpu.async_copy(h3_hbm.at[gidx_a], rows_a, sem_ga)
        cp_a0.wait()

        def pair(p, carry2):
            ia = 2 * p
            ib = 2 * p + 1
            _load_meta(ib, slab, head, src_b, dst_b, gidx_b, al_b)
            cp_gb = pltpu.async_copy(h3_hbm.at[gidx_b], rows_b, sem_gb)
            _scale(rows_a, al_a)
            cp_sa = pltpu.async_copy(rows_a, acc_sh.at[dst_a], sem_sa,
                                     add=True)
            cp_gb.wait()
            _scale(rows_b, al_b)
            cp_sa.wait()

            @pl.when(ia + 2 < NCH_S)
            def _():
                _load_meta(ia + 2, slab, head, src_a, dst_a, gidx_a, al_a)
                cp_ga = pltpu.async_copy(h3_hbm.at[gidx_a], rows_a, sem_ga)
                pltpu.sync_copy(rows_b, acc_sh.at[dst_b], add=True)
                cp_ga.wait()

            @pl.when(ia + 2 >= NCH_S)
            def _():
                pltpu.sync_copy(rows_b, acc_sh.at[dst_b], add=True)
            return carry2
        lax.fori_loop(0, NCH_S // 2, pair, 0)
        plsc.subcore_barrier()

        pltpu.sync_copy(acc_sh.at[pl.ds(s * RPS, RPS)],
                        out_hbm.at[pl.ds(slab * NPAD + s * RPS, RPS)])
        return carry
    lax.fori_loop(0, 8, slab_loop, 0)


def _mm2_body(o1_ref, w2_ref, b1_ref, asw_ref, adw_ref,
              h2_ref, a2s_ref, a2d_ref):
    acc = jnp.zeros((BLK, CLS), jnp.float32)
    for sl in range(16):
        hb = o1_ref[sl] + b1_ref[sl:sl + 1, :]
        hb = jnp.where(hb > 0, hb, jnp.exp(jnp.minimum(hb, 0.0)) - 1.0)
        acc = acc + jnp.dot(hb, w2_ref[sl], preferred_element_type=jnp.float32)
    h2_ref[...] = acc
    a2s = jnp.sum(acc * asw_ref[...], axis=1, keepdims=True)
    a2d = jnp.sum(acc * adw_ref[...], axis=1, keepdims=True)
    a2s_ref[...] = jnp.broadcast_to(a2s, (BLK, 16))
    a2d_ref[...] = jnp.broadcast_to(a2d, (BLK, 16))


def _mm2(out1, W2r, b1r, att_src2, att_dst2):
    return pl.pallas_call(
        _mm2_body,
        grid=(NBLK,),
        in_specs=[
            pl.BlockSpec((16, BLK, 128), lambda i: (0, i, 0)),
            pl.BlockSpec((16, 128, CLS), lambda i: (0, 0, 0)),
            pl.BlockSpec((16, 128), lambda i: (0, 0)),
            pl.BlockSpec((1, CLS), lambda i: (0, 0)),
            pl.BlockSpec((1, CLS), lambda i: (0, 0)),
        ],
        out_specs=[
            pl.BlockSpec((BLK, CLS), lambda i: (i, 0)),
            pl.BlockSpec((BLK, 16), lambda i: (i, 0)),
            pl.BlockSpec((BLK, 16), lambda i: (i, 0)),
        ],
        out_shape=[
            jax.ShapeDtypeStruct((NPAD, CLS), jnp.float32),
            jax.ShapeDtypeStruct((NPAD, 16), jnp.float32),
            jax.ShapeDtypeStruct((NPAD, 16), jnp.float32),
        ],
    )(out1, W2r, b1r, att_src2, att_dst2)


@functools.partial(
    pl.kernel,
    out_type=[
        jax.ShapeDtypeStruct((NPAD, 16), jnp.float32),
        jax.ShapeDtypeStruct((NPAD, 16), jnp.float32),
    ],
    mesh=_MESH,
    compiler_params=_SC_PARAMS,
    scratch_types=[
        pltpu.VMEM((CH,), jnp.int32),
        pltpu.VMEM((CH,), jnp.int32),
        pltpu.VMEM((CH, 16), jnp.float32),
        pltpu.VMEM((CH, 16), jnp.float32),
        pltpu.VMEM((CH, 16), jnp.float32),
        pltpu.VMEM((CH, 16), jnp.float32),
        pltpu.VMEM((RPS, 16), jnp.float32),
        pltpu.VMEM_SHARED((NPAD, 16), jnp.float32),
        pltpu.SemaphoreType.DMA,
        pltpu.SemaphoreType.DMA,
        pltpu.SemaphoreType.DMA,
    ],
)
def _msg2(h2_hbm, src_hbm, dst_hbm, s_hbm, d0_hbm, d1_hbm,
          o0_out, o1_out,
          src_v, dst_v, s_blk, dr0, dr1, rows, zbuf, acc_sh,
          sem1, sem2, sem3):
    c = lax.axis_index("c")
    s = lax.axis_index("s")
    wid = c * NS + s

    def zrow(i, carry):
        zbuf[i, :] = jnp.zeros((L,), jnp.float32)
        return carry
    lax.fori_loop(0, RPS, zrow, 0)
    pltpu.sync_copy(zbuf, acc_sh.at[pl.ds(s * RPS, RPS)])
    plsc.subcore_barrier()

    def chunk(i, carry):
        base = wid * EPT + i * CH
        pltpu.sync_copy(src_hbm.at[pl.ds(base, CH)], src_v)
        pltpu.sync_copy(dst_hbm.at[pl.ds(base, CH)], dst_v)
        pltpu.sync_copy(s_hbm.at[pl.ds(base, CH)], s_blk)
        cp1 = pltpu.async_copy(d0_hbm.at[dst_v], dr0, sem1)
        cp2 = pltpu.async_copy(d1_hbm.at[dst_v], dr1, sem2)
        cp3 = pltpu.async_copy(h2_hbm.at[src_v], rows, sem3)
        cp1.wait()
        cp2.wait()
        cp3.wait()

        def row(e, carry2):
            alpha = s_blk[e, :] / (dr0[e, :] + dr1[e, :])
            rows[e, :] = rows[e, :] * alpha
            return carry2
        lax.fori_loop(0, CH, row, 0)

        pltpu.sync_copy(rows, acc_sh.at[dst_v], add=True)
        return carry
    lax.fori_loop(0, NCH_W, chunk, 0)
    plsc.subcore_barrier()

    @pl.when(c == 0)
    def _():
        pltpu.sync_copy(acc_sh.at[pl.ds(s * RPS, RPS)],
                        o0_out.at[pl.ds(s * RPS, RPS)])

    @pl.when(c == 1)
    def _():
        pltpu.sync_copy(acc_sh.at[pl.ds(s * RPS, RPS)],
                        o1_out.at[pl.ds(s * RPS, RPS)])


def _final_body(p0_ref, p1_ref, b2_ref, o_ref):
    logits = p0_ref[...] + p1_ref[...] + b2_ref[...]
    m = jnp.max(logits, axis=1, keepdims=True)
    ex = jnp.exp(logits - m)
    lse = jnp.log(jnp.sum(ex, axis=1, keepdims=True))
    o_ref[...] = logits - m - lse


def _final(o0, o1, b2r):
    return pl.pallas_call(
        _final_body,
        grid=(NBLK,),
        in_specs=[
            pl.BlockSpec((BLK, CLS), lambda i: (i, 0)),
            pl.BlockSpec((BLK, CLS), lambda i: (i, 0)),
            pl.BlockSpec((1, CLS), lambda i: (0, 0)),
        ],
        out_specs=pl.BlockSpec((BLK, CLS), lambda i: (i, 0)),
        out_shape=jax.ShapeDtypeStruct((NPAD, CLS), jnp.float32),
    )(o0, o1, b2r)


def kernel(x, edge_index, W1, att_src1, att_dst1, b1, W2, att_src2, att_dst2, b2):
    xp = jnp.concatenate(
        [x.astype(jnp.float32), jnp.zeros((NPAD - N, F), jnp.float32)])
    loop = jnp.arange(N, dtype=jnp.int32)
    pad = EPAD - E
    src = jnp.concatenate([edge_index[0].astype(jnp.int32), loop,
                           jnp.zeros((pad,), jnp.int32)])
    dst = jnp.concatenate([edge_index[1].astype(jnp.int32), loop,
                           jnp.full((pad,), N, jnp.int32)])

    h3, asrc1, adst1 = _mm1(xp, W1, att_src1, att_dst1)
    s1, d10, d11 = _edge_softmax(asrc1, adst1, src, dst)
    al1 = _alpha1(s1, dst, d10, d11)
    out1f = _msg1(h3.reshape(16 * NPAD, 128), src, dst, al1)

    h2, a2s, a2d = _mm2(out1f.reshape(16, NPAD, 128),
                        W2.reshape(16, 128, CLS), b1.reshape(16, 128),
                        att_src2, att_dst2)
    s2, d20, d21 = _edge_softmax(a2s, a2d, src, dst)
    o20, o21 = _msg2(h2, src, dst, s2, d20, d21)

    out = _final(o20, o21, b2.reshape(1, CLS))
    return out[:N]


# speedup vs baseline: 8.5841x; 1.0235x over previous
import functools

import jax
import jax.numpy as jnp
from jax import lax
from jax.experimental import pallas as pl
from jax.experimental.pallas import tpu as pltpu
from jax.experimental.pallas import tpu_sc as plsc

N = 10000
F = 256
HID = 256
H = 8
CLS = 16
E0 = 160000

NC, NS, L = 2, 16, 16
NW = NC * NS

NPAD = 10240
BLK = 320
NBLK = NPAD // BLK
RPS = NPAD // NS

E = E0 + N
EPT = 5376
EPAD = EPT * NW
CH = 128
NCH_W = EPT // CH
EPT_S = EPAD // NS
NCH_S = EPT_S // CH

_SC_PARAMS = pltpu.CompilerParams(needs_layout_passes=False,
                                  use_tc_tiling_on_sc=False)
_MESH = plsc.VectorSubcoreMesh(core_axis_name="c", subcore_axis_name="s")


def _mm1_body(x_ref, w_ref, asw_ref, adw_ref, h3_ref, asrc_ref, adst_ref):
    hb = jnp.dot(x_ref[...], w_ref[...], preferred_element_type=jnp.float32)
    for s in range(16):
        h3_ref[s, :, :] = hb[:, s * 128:(s + 1) * 128]
    for h in range(H):
        seg = hb[:, h * HID:(h + 1) * HID]
        asrc_ref[:, h:h + 1] = jnp.sum(seg * asw_ref[h:h + 1, :], axis=1,
                                       keepdims=True)
        adst_ref[:, h:h + 1] = jnp.sum(seg * adw_ref[h:h + 1, :], axis=1,
                                       keepdims=True)
    asrc_ref[:, H:] = jnp.zeros((BLK, 16 - H), jnp.float32)
    adst_ref[:, H:] = jnp.zeros((BLK, 16 - H), jnp.float32)


def _mm1(xp, W1, att_src1, att_dst1):
    return pl.pallas_call(
        _mm1_body,
        grid=(NBLK,),
        in_specs=[
            pl.BlockSpec((BLK, F), lambda i: (i, 0)),
            pl.BlockSpec((F, H * HID), lambda i: (0, 0)),
            pl.BlockSpec((H, HID), lambda i: (0, 0)),
            pl.BlockSpec((H, HID), lambda i: (0, 0)),
        ],
        out_specs=[
            pl.BlockSpec((16, BLK, 128), lambda i: (0, i, 0)),
            pl.BlockSpec((BLK, 16), lambda i: (i, 0)),
            pl.BlockSpec((BLK, 16), lambda i: (i, 0)),
        ],
        out_shape=[
            jax.ShapeDtypeStruct((16, NPAD, 128), jnp.float32),
            jax.ShapeDtypeStruct((NPAD, 16), jnp.float32),
            jax.ShapeDtypeStruct((NPAD, 16), jnp.float32),
        ],
    )(xp, W1, att_src1, att_dst1)


@functools.partial(
    pl.kernel,
    out_type=[
        jax.ShapeDtypeStruct((EPAD, 16), jnp.float32),
        jax.ShapeDtypeStruct((NPAD, 16), jnp.float32),
        jax.ShapeDtypeStruct((NPAD, 16), jnp.float32),
    ],
    mesh=_MESH,
    compiler_params=_SC_PARAMS,
    scratch_types=[
        pltpu.VMEM((CH,), jnp.int32),
        pltpu.VMEM((CH,), jnp.int32),
        pltpu.VMEM((CH, 16), jnp.float32),
        pltpu.VMEM((CH, 16), jnp.float32),
        pltpu.VMEM((CH, 16), jnp.float32),
        pltpu.VMEM((RPS, 16), jnp.float32),
        pltpu.VMEM_SHARED((NPAD, 16), jnp.float32),
        pltpu.SemaphoreType.DMA,
        pltpu.SemaphoreType.DMA,
    ],
)
def _edge_softmax(asrc_hbm, adst_hbm, src_hbm, dst_hbm,
                  s_out, d0_out, d1_out,
                  src_v, dst_v, asr, adr, s_blk, zbuf, den_sh, sem1, sem2):
    c = lax.axis_index("c")
    s = lax.axis_index("s")
    wid = c * NS + s

    def zrow(i, carry):
        zbuf[i, :] = jnp.zeros((L,), jnp.float32)
        return carry
    lax.fori_loop(0, RPS, zrow, 0)
    pltpu.sync_copy(zbuf, den_sh.at[pl.ds(s * RPS, RPS)])
    plsc.subcore_barrier()

    def chunk(i, carry):
        base = wid * EPT + i * CH
        pltpu.sync_copy(src_hbm.at[pl.ds(base, CH)], src_v)
        pltpu.sync_copy(dst_hbm.at[pl.ds(base, CH)], dst_v)
        cp1 = pltpu.async_copy(asrc_hbm.at[src_v], asr, sem1)
        cp2 = pltpu.async_copy(adst_hbm.at[dst_v], adr, sem2)
        cp1.wait()
        cp2.wait()

        def row(j, carry2):
            e = asr[j, :] + adr[j, :]
            e = jnp.maximum(e, 0.2 * e)
            s_blk[j, :] = jnp.exp(e)
            return carry2
        lax.fori_loop(0, CH, row, 0)

        pltpu.sync_copy(s_blk, s_out.at[pl.ds(base, CH)])
        pltpu.sync_copy(s_blk, den_sh.at[dst_v], add=True)
        return carry
    lax.fori_loop(0, NCH_W, chunk, 0)
    plsc.subcore_barrier()

    @pl.when(c == 0)
    def _():
        pltpu.sync_copy(den_sh.at[pl.ds(s * RPS, RPS)],
                        d0_out.at[pl.ds(s * RPS, RPS)])

    @pl.when(c == 1)
    def _():
        pltpu.sync_copy(den_sh.at[pl.ds(s * RPS, RPS)],
                        d1_out.at[pl.ds(s * RPS, RPS)])


@functools.partial(
    pl.kernel,
    out_type=jax.ShapeDtypeStruct((8 * EPAD,), jnp.float32),
    mesh=_MESH,
    compiler_params=_SC_PARAMS,
    scratch_types=[
        pltpu.VMEM((CH,), jnp.int32),
        pltpu.VMEM((CH, 16), jnp.float32),
        pltpu.VMEM((CH, 16), jnp.float32),
        pltpu.VMEM((CH, 16), jnp.float32),
        pltpu.VMEM((8, CH), jnp.float32),
        pltpu.SemaphoreType.DMA,
        pltpu.SemaphoreType.DMA,
    ],
)
def _alpha1(s_hbm, dst_hbm, d0_hbm, d1_hbm, al_out,
            dst_v, s_blk, dr0, dr1, al8, sem1, sem2):
    c = lax.axis_index("c")
    s = lax.axis_index("s")
    wid = c * NS + s

    def chunk(i, carry):
        base = wid * EPT + i * CH
        pltpu.sync_copy(dst_hbm.at[pl.ds(base, CH)], dst_v)
        pltpu.sync_copy(s_hbm.at[pl.ds(base, CH)], s_blk)
        cp1 = pltpu.async_copy(d0_hbm.at[dst_v], dr0, sem1)
        cp2 = pltpu.async_copy(d1_hbm.at[dst_v], dr1, sem2)
        cp1.wait()
        cp2.wait()
        for h in range(H):
            hv = jnp.full((L,), h, jnp.int32)
            for g in range(CH // L):
                ev = lax.iota(jnp.int32, L) + g * L
                sc = plsc.load_gather(s_blk, [ev, hv])
                dc0 = plsc.load_gather(dr0, [ev, hv])
                dc1 = plsc.load_gather(dr1, [ev, hv])
                al8[h, pl.ds(g * L, L)] = sc / (dc0 + dc1)
        for h in range(H):
            pltpu.sync_copy(al8.at[h], al_out.at[pl.ds(h * EPAD + base, CH)])
        return carry
    lax.fori_loop(0, NCH_W, chunk, 0)


@functools.partial(
    pl.kernel,
    out_type=jax.ShapeDtypeStruct((16 * NPAD, 128), jnp.float32),
    mesh=_MESH,
    compiler_params=_SC_PARAMS,
    scratch_types=[
        pltpu.VMEM((CH,), jnp.int32),
        pltpu.VMEM((CH,), jnp.int32),
        pltpu.VMEM((CH,), jnp.int32),
        pltpu.VMEM((CH,), jnp.int32),
        pltpu.VMEM((CH,), jnp.int32),
        pltpu.VMEM((CH,), jnp.int32),
        pltpu.VMEM((CH,), jnp.float32),
        pltpu.VMEM((CH,), jnp.float32),
        pltpu.VMEM((CH, 128), jnp.float32),
        pltpu.VMEM((CH, 128), jnp.float32),
        pltpu.VMEM((64, 128), jnp.float32),
        pltpu.VMEM_SHARED((NPAD, 128), jnp.float32),
        pltpu.SemaphoreType.DMA,
        pltpu.SemaphoreType.DMA,
        pltpu.SemaphoreType.DMA,
        pltpu.SemaphoreType.DMA,
    ],
)
def _msg1(h3_hbm, src_hbm, dst_hbm, al_hbm, out_hbm,
          src_a, src_b, dst_a, dst_b, gidx_a, gidx_b, al_a, al_b,
          rows_a, rows_b, zbuf, acc_sh, sem_ga, sem_gb, sem_sa, sem_sb):
    c = lax.axis_index("c")
    s = lax.axis_index("s")

    def zrow(i, carry):
        for k in range(128 // L):
            zbuf[i, pl.ds(k * L, L)] = jnp.zeros((L,), jnp.float32)
        return carry
    lax.fori_loop(0, 64, zrow, 0)

    def _load_meta(i, slab, head, src_v, dst_v, gidx, al):
        base = s * EPT_S + i * CH
        pltpu.sync_copy(src_hbm.at[pl.ds(base, CH)], src_v)
        for g in range(CH // L):
            gidx[pl.ds(g * L, L)] = src_v[pl.ds(g * L, L)] + slab * NPAD
        pltpu.sync_copy(dst_hbm.at[pl.ds(base, CH)], dst_v)
        pltpu.sync_copy(al_hbm.at[pl.ds(head * EPAD + base, CH)], al)

    def _scale(rows, al):
        @plsc.parallel_loop(0, CH, 1, unroll=4)
        def _(e):
            av = plsc.load_gather(al, [jnp.full((L,), e, jnp.int32)])
            for k in range(128 // L):
                rows[e, pl.ds(k * L, L)] = rows[e, pl.ds(k * L, L)] * av

    def slab_loop(j, carry):
        slab = c * 8 + j
        head = slab // 2

        def zcp(k, carry2):
            pltpu.sync_copy(zbuf, acc_sh.at[pl.ds(s * RPS + k * 64, 64)])
            return carry2
        lax.fori_loop(0, RPS // 64, zcp, 0)
        plsc.subcore_barrier()

        _load_meta(0, slab, head, src_a, dst_a, gidx_a, al_a)
        cp_a0 = pltpu.async_copy(h3_hbm.at[gidx_a], rows_a, sem_ga)
        cp_a0.wait()

        def pair(p, carry2):
            ia = 2 * p
            ib = 2 * p + 1
            _load_meta(ib, slab, head, src_b, dst_b, gidx_b, al_b)
            cp_gb = pltpu.async_copy(h3_hbm.at[gidx_b], rows_b, sem_gb)
            _scale(rows_a, al_a)
            cp_sa = pltpu.async_copy(rows_a, acc_sh.at[dst_a], sem_sa,
                                     add=True)
            cp_gb.wait()
            _scale(rows_b, al_b)
            cp_sa.wait()

            @pl.when(ia + 2 < NCH_S)
            def _():
                _load_meta(ia + 2, slab, head, src_a, dst_a, gidx_a, al_a)
                cp_ga = pltpu.async_copy(h3_hbm.at[gidx_a], rows_a, sem_ga)
                pltpu.sync_copy(rows_b, acc_sh.at[dst_b], add=True)
                cp_ga.wait()

            @pl.when(ia + 2 >= NCH_S)
            def _():
                pltpu.sync_copy(rows_b, acc_sh.at[dst_b], add=True)
            return carry2
        lax.fori_loop(0, NCH_S // 2, pair, 0)
        plsc.subcore_barrier()

        pltpu.sync_copy(acc_sh.at[pl.ds(s * RPS, RPS)],
                        out_hbm.at[pl.ds(slab * NPAD + s * RPS, RPS)])
        return carry
    lax.fori_loop(0, 8, slab_loop, 0)


def _mm2_body(o1_ref, w2_ref, b1_ref, asw_ref, adw_ref,
              h2_ref, a2s_ref, a2d_ref):
    acc = jnp.zeros((BLK, CLS), jnp.float32)
    for sl in range(16):
        hb = o1_ref[sl] + b1_ref[sl:sl + 1, :]
        hb = jnp.where(hb > 0, hb, jnp.exp(jnp.minimum(hb, 0.0)) - 1.0)
        acc = acc + jnp.dot(hb, w2_ref[sl], preferred_element_type=jnp.float32)
    h2_ref[...] = acc
    a2s = jnp.sum(acc * asw_ref[...], axis=1, keepdims=True)
    a2d = jnp.sum(acc * adw_ref[...], axis=1, keepdims=True)
    a2s_ref[...] = jnp.broadcast_to(a2s, (BLK, 16))
    a2d_ref[...] = jnp.broadcast_to(a2d, (BLK, 16))


def _mm2(out1, W2r, b1r, att_src2, att_dst2):
    return pl.pallas_call(
        _mm2_body,
        grid=(NBLK,),
        in_specs=[
            pl.BlockSpec((16, BLK, 128), lambda i: (0, i, 0)),
            pl.BlockSpec((16, 128, CLS), lambda i: (0, 0, 0)),
            pl.BlockSpec((16, 128), lambda i: (0, 0)),
            pl.BlockSpec((1, CLS), lambda i: (0, 0)),
            pl.BlockSpec((1, CLS), lambda i: (0, 0)),
        ],
        out_specs=[
            pl.BlockSpec((BLK, CLS), lambda i: (i, 0)),
            pl.BlockSpec((BLK, 16), lambda i: (i, 0)),
            pl.BlockSpec((BLK, 16), lambda i: (i, 0)),
        ],
        out_shape=[
            jax.ShapeDtypeStruct((NPAD, CLS), jnp.float32),
            jax.ShapeDtypeStruct((NPAD, 16), jnp.float32),
            jax.ShapeDtypeStruct((NPAD, 16), jnp.float32),
        ],
    )(out1, W2r, b1r, att_src2, att_dst2)


@functools.partial(
    pl.kernel,
    out_type=[
        jax.ShapeDtypeStruct((NPAD, 16), jnp.float32),
        jax.ShapeDtypeStruct((NPAD, 16), jnp.float32),
    ],
    mesh=_MESH,
    compiler_params=_SC_PARAMS,
    scratch_types=[
        pltpu.VMEM((CH,), jnp.int32),
        pltpu.VMEM((CH,), jnp.int32),
        pltpu.VMEM((CH, 16), jnp.float32),
        pltpu.VMEM((CH, 16), jnp.float32),
        pltpu.VMEM((CH, 16), jnp.float32),
        pltpu.VMEM((CH, 16), jnp.float32),
        pltpu.VMEM((RPS, 16), jnp.float32),
        pltpu.VMEM_SHARED((NPAD, 16), jnp.float32),
        pltpu.SemaphoreType.DMA,
        pltpu.SemaphoreType.DMA,
        pltpu.SemaphoreType.DMA,
    ],
)
def _msg2(h2_hbm, src_hbm, dst_hbm, s_hbm, d0_hbm, d1_hbm,
          o0_out, o1_out,
          src_v, dst_v, s_blk, dr0, dr1, rows, zbuf, acc_sh,
          sem1, sem2, sem3):
    c = lax.axis_index("c")
    s = lax.axis_index("s")
    wid = c * NS + s

    def zrow(i, carry):
        zbuf[i, :] = jnp.zeros((L,), jnp.float32)
        return carry
    lax.fori_loop(0, RPS, zrow, 0)
    pltpu.sync_copy(zbuf, acc_sh.at[pl.ds(s * RPS, RPS)])
    plsc.subcore_barrier()

    def chunk(i, carry):
        base = wid * EPT + i * CH
        pltpu.sync_copy(src_hbm.at[pl.ds(base, CH)], src_v)
        pltpu.sync_copy(dst_hbm.at[pl.ds(base, CH)], dst_v)
        pltpu.sync_copy(s_hbm.at[pl.ds(base, CH)], s_blk)
        cp1 = pltpu.async_copy(d0_hbm.at[dst_v], dr0, sem1)
        cp2 = pltpu.async_copy(d1_hbm.at[dst_v], dr1, sem2)
        cp3 = pltpu.async_copy(h2_hbm.at[src_v], rows, sem3)
        cp1.wait()
        cp2.wait()
        cp3.wait()

        def row(e, carry2):
            alpha = s_blk[e, :] / (dr0[e, :] + dr1[e, :])
            rows[e, :] = rows[e, :] * alpha
            return carry2
        lax.fori_loop(0, CH, row, 0)

        pltpu.sync_copy(rows, acc_sh.at[dst_v], add=True)
        return carry
    lax.fori_loop(0, NCH_W, chunk, 0)
    plsc.subcore_barrier()

    @pl.when(c == 0)
    def _():
        pltpu.sync_copy(acc_sh.at[pl.ds(s * RPS, RPS)],
                        o0_out.at[pl.ds(s * RPS, RPS)])

    @pl.when(c == 1)
    def _():
        pltpu.sync_copy(acc_sh.at[pl.ds(s * RPS, RPS)],
                        o1_out.at[pl.ds(s * RPS, RPS)])


def _final_body(p0_ref, p1_ref, b2_ref, o_ref):
    logits = p0_ref[...] + p1_ref[...] + b2_ref[...]
    m = jnp.max(logits, axis=1, keepdims=True)
    ex = jnp.exp(logits - m)
    lse = jnp.log(jnp.sum(ex, axis=1, keepdims=True))
    o_ref[...] = logits - m - lse


def _final(o0, o1, b2r):
    return pl.pallas_call(
        _final_body,
        grid=(NBLK,),
        in_specs=[
            pl.BlockSpec((BLK, CLS), lambda i: (i, 0)),
            pl.BlockSpec((BLK, CLS), lambda i: (i, 0)),
            pl.BlockSpec((1, CLS), lambda i: (0, 0)),
        ],
        out_specs=pl.BlockSpec((BLK, CLS), lambda i: (i, 0)),
        out_shape=jax.ShapeDtypeStruct((NPAD, CLS), jnp.float32),
    )(o0, o1, b2r)


def kernel(x, edge_index, W1, att_src1, att_dst1, b1, W2, att_src2, att_dst2, b2):
    xp = jnp.concatenate(
        [x.astype(jnp.float32), jnp.zeros((NPAD - N, F), jnp.float32)])
    loop = jnp.arange(N, dtype=jnp.int32)
    pad = EPAD - E
    src = jnp.concatenate([edge_index[0].astype(jnp.int32), loop,
                           jnp.zeros((pad,), jnp.int32)])
    dst = jnp.concatenate([edge_index[1].astype(jnp.int32), loop,
                           jnp.full((pad,), N, jnp.int32)])

    h3, asrc1, adst1 = _mm1(xp, W1, att_src1, att_dst1)
    s1, d10, d11 = _edge_softmax(asrc1, adst1, src, dst)
    al1 = _alpha1(s1, dst, d10, d11)
    out1f = _msg1(h3.reshape(16 * NPAD, 128), src, dst, al1)

    h2, a2s, a2d = _mm2(out1f.reshape(16, NPAD, 128),
                        W2.reshape(16, 128, CLS), b1.reshape(16, 128),
                        att_src2, att_dst2)
    s2, d20, d21 = _edge_softmax(a2s, a2d, src, dst)
    o20, o21 = _msg2(h2, src, dst, s2, d20, d21)

    out = _final(o20, o21, b2.reshape(1, CLS))
    return out[:N]


# speedup vs baseline: 10.3119x; 1.2013x over previous
import functools

import jax
import jax.numpy as jnp
from jax import lax
from jax.experimental import pallas as pl
from jax.experimental.pallas import tpu as pltpu
from jax.experimental.pallas import tpu_sc as plsc

N = 10000
F = 256
HID = 256
H = 8
CLS = 16
E0 = 160000

NC, NS, L = 2, 16, 16
NW = NC * NS

NPAD = 10240
BLK = 320
NBLK = NPAD // BLK
RPS = NPAD // NS

E = E0 + N
EPT = 5376
EPAD = EPT * NW
CH = 128
NCH_W = EPT // CH
EPT_S = EPAD // NS
NCH_S = EPT_S // CH

_SC_PARAMS = pltpu.CompilerParams(needs_layout_passes=False,
                                  use_tc_tiling_on_sc=False)
_MESH = plsc.VectorSubcoreMesh(core_axis_name="c", subcore_axis_name="s")


def _mm1_body(x_ref, w_ref, asw_ref, adw_ref, h3_ref, asrc_ref, adst_ref):
    hb = jnp.dot(x_ref[...], w_ref[...], preferred_element_type=jnp.float32)
    for s in range(16):
        h3_ref[s, :, :] = hb[:, s * 128:(s + 1) * 128].astype(jnp.bfloat16)
    for h in range(H):
        seg = hb[:, h * HID:(h + 1) * HID]
        asrc_ref[:, h:h + 1] = jnp.sum(seg * asw_ref[h:h + 1, :], axis=1,
                                       keepdims=True)
        adst_ref[:, h:h + 1] = jnp.sum(seg * adw_ref[h:h + 1, :], axis=1,
                                       keepdims=True)
    asrc_ref[:, H:] = jnp.zeros((BLK, 16 - H), jnp.float32)
    adst_ref[:, H:] = jnp.zeros((BLK, 16 - H), jnp.float32)


def _mm1(xp, W1, att_src1, att_dst1):
    return pl.pallas_call(
        _mm1_body,
        grid=(NBLK,),
        in_specs=[
            pl.BlockSpec((BLK, F), lambda i: (i, 0)),
            pl.BlockSpec((F, H * HID), lambda i: (0, 0)),
            pl.BlockSpec((H, HID), lambda i: (0, 0)),
            pl.BlockSpec((H, HID), lambda i: (0, 0)),
        ],
        out_specs=[
            pl.BlockSpec((16, BLK, 128), lambda i: (0, i, 0)),
            pl.BlockSpec((BLK, 16), lambda i: (i, 0)),
            pl.BlockSpec((BLK, 16), lambda i: (i, 0)),
        ],
        out_shape=[
            jax.ShapeDtypeStruct((16, NPAD, 128), jnp.bfloat16),
            jax.ShapeDtypeStruct((NPAD, 16), jnp.float32),
            jax.ShapeDtypeStruct((NPAD, 16), jnp.float32),
        ],
    )(xp, W1, att_src1, att_dst1)


@functools.partial(
    pl.kernel,
    out_type=[
        jax.ShapeDtypeStruct((EPAD, 16), jnp.float32),
        jax.ShapeDtypeStruct((NPAD, 16), jnp.float32),
        jax.ShapeDtypeStruct((NPAD, 16), jnp.float32),
    ],
    mesh=_MESH,
    compiler_params=_SC_PARAMS,
    scratch_types=[
        pltpu.VMEM((CH,), jnp.int32),
        pltpu.VMEM((CH,), jnp.int32),
        pltpu.VMEM((CH, 16), jnp.float32),
        pltpu.VMEM((CH, 16), jnp.float32),
        pltpu.VMEM((CH, 16), jnp.float32),
        pltpu.VMEM((RPS, 16), jnp.float32),
        pltpu.VMEM_SHARED((NPAD, 16), jnp.float32),
        pltpu.SemaphoreType.DMA,
        pltpu.SemaphoreType.DMA,
    ],
)
def _edge_softmax(asrc_hbm, adst_hbm, src_hbm, dst_hbm,
                  s_out, d0_out, d1_out,
                  src_v, dst_v, asr, adr, s_blk, zbuf, den_sh, sem1, sem2):
    c = lax.axis_index("c")
    s = lax.axis_index("s")
    wid = c * NS + s

    def zrow(i, carry):
        zbuf[i, :] = jnp.zeros((L,), jnp.float32)
        return carry
    lax.fori_loop(0, RPS, zrow, 0)
    pltpu.sync_copy(zbuf, den_sh.at[pl.ds(s * RPS, RPS)])
    plsc.subcore_barrier()

    def chunk(i, carry):
        base = wid * EPT + i * CH
        pltpu.sync_copy(src_hbm.at[pl.ds(base, CH)], src_v)
        pltpu.sync_copy(dst_hbm.at[pl.ds(base, CH)], dst_v)
        cp1 = pltpu.async_copy(asrc_hbm.at[src_v], asr, sem1)
        cp2 = pltpu.async_copy(adst_hbm.at[dst_v], adr, sem2)
        cp1.wait()
        cp2.wait()

        def row(j, carry2):
            e = asr[j, :] + adr[j, :]
            e = jnp.maximum(e, 0.2 * e)
            s_blk[j, :] = jnp.exp(e)
            return carry2
        lax.fori_loop(0, CH, row, 0)

        pltpu.sync_copy(s_blk, s_out.at[pl.ds(base, CH)])
        pltpu.sync_copy(s_blk, den_sh.at[dst_v], add=True)
        return carry
    lax.fori_loop(0, NCH_W, chunk, 0)
    plsc.subcore_barrier()

    @pl.when(c == 0)
    def _():
        pltpu.sync_copy(den_sh.at[pl.ds(s * RPS, RPS)],
                        d0_out.at[pl.ds(s * RPS, RPS)])

    @pl.when(c == 1)
    def _():
        pltpu.sync_copy(den_sh.at[pl.ds(s * RPS, RPS)],
                        d1_out.at[pl.ds(s * RPS, RPS)])


@functools.partial(
    pl.kernel,
    out_type=jax.ShapeDtypeStruct((8 * EPAD,), jnp.float32),
    mesh=_MESH,
    compiler_params=_SC_PARAMS,
    scratch_types=[
        pltpu.VMEM((CH,), jnp.int32),
        pltpu.VMEM((CH, 16), jnp.float32),
        pltpu.VMEM((CH, 16), jnp.float32),
        pltpu.VMEM((CH, 16), jnp.float32),
        pltpu.VMEM((8, CH), jnp.float32),
        pltpu.SemaphoreType.DMA,
        pltpu.SemaphoreType.DMA,
    ],
)
def _alpha1(s_hbm, dst_hbm, d0_hbm, d1_hbm, al_out,
            dst_v, s_blk, dr0, dr1, al8, sem1, sem2):
    c = lax.axis_index("c")
    s = lax.axis_index("s")
    wid = c * NS + s

    def chunk(i, carry):
        base = wid * EPT + i * CH
        pltpu.sync_copy(dst_hbm.at[pl.ds(base, CH)], dst_v)
        pltpu.sync_copy(s_hbm.at[pl.ds(base, CH)], s_blk)
        cp1 = pltpu.async_copy(d0_hbm.at[dst_v], dr0, sem1)
        cp2 = pltpu.async_copy(d1_hbm.at[dst_v], dr1, sem2)
        cp1.wait()
        cp2.wait()
        for h in range(H):
            hv = jnp.full((L,), h, jnp.int32)
            for g in range(CH // L):
                ev = lax.iota(jnp.int32, L) + g * L
                sc = plsc.load_gather(s_blk, [ev, hv])
                dc0 = plsc.load_gather(dr0, [ev, hv])
                dc1 = plsc.load_gather(dr1, [ev, hv])
                al8[h, pl.ds(g * L, L)] = sc / (dc0 + dc1)
        for h in range(H):
            pltpu.sync_copy(al8.at[h], al_out.at[pl.ds(h * EPAD + base, CH)])
        return carry
    lax.fori_loop(0, NCH_W, chunk, 0)


@functools.partial(
    pl.kernel,
    out_type=jax.ShapeDtypeStruct((16 * NPAD, 128), jnp.float32),
    mesh=_MESH,
    compiler_params=_SC_PARAMS,
    scratch_types=[
        pltpu.VMEM((CH,), jnp.int32),
        pltpu.VMEM((CH,), jnp.int32),
        pltpu.VMEM((CH,), jnp.int32),
        pltpu.VMEM((CH,), jnp.int32),
        pltpu.VMEM((CH,), jnp.int32),
        pltpu.VMEM((CH,), jnp.int32),
        pltpu.VMEM((CH,), jnp.float32),
        pltpu.VMEM((CH,), jnp.float32),
        pltpu.VMEM((CH, 128), jnp.bfloat16),
        pltpu.VMEM((CH, 128), jnp.bfloat16),
        pltpu.VMEM((CH, 128), jnp.float32),
        pltpu.VMEM((16, 128), jnp.float32),
        pltpu.VMEM_SHARED((NPAD, 128), jnp.float32),
        pltpu.SemaphoreType.DMA,
        pltpu.SemaphoreType.DMA,
    ],
)
def _msg1(h3_hbm, src_hbm, dst_hbm, al_hbm, out_hbm,
          src_a, src_b, dst_a, dst_b, gidx_a, gidx_b, al_a, al_b,
          rows_a, rows_b, rows32, zbuf, acc_sh, sem_ga, sem_gb):
    c = lax.axis_index("c")
    s = lax.axis_index("s")

    def zrow(i, carry):
        for k in range(128 // L):
            zbuf[i, pl.ds(k * L, L)] = jnp.zeros((L,), jnp.float32)
        return carry
    lax.fori_loop(0, 16, zrow, 0)

    def _load_meta(i, slab, head, src_v, dst_v, gidx, al):
        base = s * EPT_S + i * CH
        pltpu.sync_copy(src_hbm.at[pl.ds(base, CH)], src_v)
        for g in range(CH // L):
            gidx[pl.ds(g * L, L)] = src_v[pl.ds(g * L, L)] + slab * NPAD
        pltpu.sync_copy(dst_hbm.at[pl.ds(base, CH)], dst_v)
        pltpu.sync_copy(al_hbm.at[pl.ds(head * EPAD + base, CH)], al)

    def _scale(rows, al):
        @plsc.parallel_loop(0, CH, 1, unroll=4)
        def _(e):
            av = plsc.load_gather(al, [jnp.full((L,), e, jnp.int32)])
            for k in range(128 // 32):
                x = rows[e, pl.ds(k * 32, 32)]
                u0, u1 = plsc.unpack(x, format=plsc.PackFormat.INTERLEAVED)
                rows32[e, pl.ds(k * 32, L)] = u0 * av
                rows32[e, pl.ds(k * 32 + L, L)] = u1 * av

    def slab_loop(j, carry):
        slab = c * 8 + j
        head = slab // 2

        def zcp(k, carry2):
            pltpu.sync_copy(zbuf, acc_sh.at[pl.ds(s * RPS + k * 16, 16)])
            return carry2
        lax.fori_loop(0, RPS // 16, zcp, 0)
        plsc.subcore_barrier()

        _load_meta(0, slab, head, src_a, dst_a, gidx_a, al_a)
        cp_a0 = pltpu.async_copy(h3_hbm.at[gidx_a], rows_a, sem_ga)
        cp_a0.wait()

        def pair(p, carry2):
            ia = 2 * p
            ib = 2 * p + 1
            _load_meta(ib, slab, head, src_b, dst_b, gidx_b, al_b)
            cp_gb = pltpu.async_copy(h3_hbm.at[gidx_b], rows_b, sem_gb)
            _scale(rows_a, al_a)
            pltpu.sync_copy(rows32, acc_sh.at[dst_a], add=True)
            cp_gb.wait()

            @pl.when(ia + 2 < NCH_S)
            def _():
                _load_meta(ia + 2, slab, head, src_a, dst_a, gidx_a, al_a)
                cp_ga = pltpu.async_copy(h3_hbm.at[gidx_a], rows_a, sem_ga)
                _scale(rows_b, al_b)
                pltpu.sync_copy(rows32, acc_sh.at[dst_b], add=True)
                cp_ga.wait()

            @pl.when(ia + 2 >= NCH_S)
            def _():
                _scale(rows_b, al_b)
                pltpu.sync_copy(rows32, acc_sh.at[dst_b], add=True)
            return carry2
        lax.fori_loop(0, NCH_S // 2, pair, 0)
        plsc.subcore_barrier()

        pltpu.sync_copy(acc_sh.at[pl.ds(s * RPS, RPS)],
                        out_hbm.at[pl.ds(slab * NPAD + s * RPS, RPS)])
        return carry
    lax.fori_loop(0, 8, slab_loop, 0)


def _mm2_body(o1_ref, w2_ref, b1_ref, asw_ref, adw_ref,
              h2_ref, a2s_ref, a2d_ref):
    acc = jnp.zeros((BLK, CLS), jnp.float32)
    for sl in range(16):
        hb = o1_ref[sl] + b1_ref[sl:sl + 1, :]
        hb = jnp.where(hb > 0, hb, jnp.exp(jnp.minimum(hb, 0.0)) - 1.0)
        acc = acc + jnp.dot(hb, w2_ref[sl], preferred_element_type=jnp.float32)
    h2_ref[...] = acc
    a2s = jnp.sum(acc * asw_ref[...], axis=1, keepdims=True)
    a2d = jnp.sum(acc * adw_ref[...], axis=1, keepdims=True)
    a2s_ref[...] = jnp.broadcast_to(a2s, (BLK, 16))
    a2d_ref[...] = jnp.broadcast_to(a2d, (BLK, 16))


def _mm2(out1, W2r, b1r, att_src2, att_dst2):
    return pl.pallas_call(
        _mm2_body,
        grid=(NBLK,),
        in_specs=[
            pl.BlockSpec((16, BLK, 128), lambda i: (0, i, 0)),
            pl.BlockSpec((16, 128, CLS), lambda i: (0, 0, 0)),
            pl.BlockSpec((16, 128), lambda i: (0, 0)),
            pl.BlockSpec((1, CLS), lambda i: (0, 0)),
            pl.BlockSpec((1, CLS), lambda i: (0, 0)),
        ],
        out_specs=[
            pl.BlockSpec((BLK, CLS), lambda i: (i, 0)),
            pl.BlockSpec((BLK, 16), lambda i: (i, 0)),
            pl.BlockSpec((BLK, 16), lambda i: (i, 0)),
        ],
        out_shape=[
            jax.ShapeDtypeStruct((NPAD, CLS), jnp.float32),
            jax.ShapeDtypeStruct((NPAD, 16), jnp.float32),
            jax.ShapeDtypeStruct((NPAD, 16), jnp.float32),
        ],
    )(out1, W2r, b1r, att_src2, att_dst2)


@functools.partial(
    pl.kernel,
    out_type=[
        jax.ShapeDtypeStruct((NPAD, 16), jnp.float32),
        jax.ShapeDtypeStruct((NPAD, 16), jnp.float32),
    ],
    mesh=_MESH,
    compiler_params=_SC_PARAMS,
    scratch_types=[
        pltpu.VMEM((CH,), jnp.int32),
        pltpu.VMEM((CH,), jnp.int32),
        pltpu.VMEM((CH, 16), jnp.float32),
        pltpu.VMEM((CH, 16), jnp.float32),
        pltpu.VMEM((CH, 16), jnp.float32),
        pltpu.VMEM((CH, 16), jnp.float32),
        pltpu.VMEM((RPS, 16), jnp.float32),
        pltpu.VMEM_SHARED((NPAD, 16), jnp.float32),
        pltpu.SemaphoreType.DMA,
        pltpu.SemaphoreType.DMA,
        pltpu.SemaphoreType.DMA,
    ],
)
def _msg2(h2_hbm, src_hbm, dst_hbm, s_hbm, d0_hbm, d1_hbm,
          o0_out, o1_out,
          src_v, dst_v, s_blk, dr0, dr1, rows, zbuf, acc_sh,
          sem1, sem2, sem3):
    c = lax.axis_index("c")
    s = lax.axis_index("s")
    wid = c * NS + s

    def zrow(i, carry):
        zbuf[i, :] = jnp.zeros((L,), jnp.float32)
        return carry
    lax.fori_loop(0, RPS, zrow, 0)
    pltpu.sync_copy(zbuf, acc_sh.at[pl.ds(s * RPS, RPS)])
    plsc.subcore_barrier()

    def chunk(i, carry):
        base = wid * EPT + i * CH
        pltpu.sync_copy(src_hbm.at[pl.ds(base, CH)], src_v)
        pltpu.sync_copy(dst_hbm.at[pl.ds(base, CH)], dst_v)
        pltpu.sync_copy(s_hbm.at[pl.ds(base, CH)], s_blk)
        cp1 = pltpu.async_copy(d0_hbm.at[dst_v], dr0, sem1)
        cp2 = pltpu.async_copy(d1_hbm.at[dst_v], dr1, sem2)
        cp3 = pltpu.async_copy(h2_hbm.at[src_v], rows, sem3)
        cp1.wait()
        cp2.wait()
        cp3.wait()

        def row(e, carry2):
            alpha = s_blk[e, :] / (dr0[e, :] + dr1[e, :])
            rows[e, :] = rows[e, :] * alpha
            return carry2
        lax.fori_loop(0, CH, row, 0)

        pltpu.sync_copy(rows, acc_sh.at[dst_v], add=True)
        return carry
    lax.fori_loop(0, NCH_W, chunk, 0)
    plsc.subcore_barrier()

    @pl.when(c == 0)
    def _():
        pltpu.sync_copy(acc_sh.at[pl.ds(s * RPS, RPS)],
                        o0_out.at[pl.ds(s * RPS, RPS)])

    @pl.when(c == 1)
    def _():
        pltpu.sync_copy(acc_sh.at[pl.ds(s * RPS, RPS)],
                        o1_out.at[pl.ds(s * RPS, RPS)])


def _final_body(p0_ref, p1_ref, b2_ref, o_ref):
    logits = p0_ref[...] + p1_ref[...] + b2_ref[...]
    m = jnp.max(logits, axis=1, keepdims=True)
    ex = jnp.exp(logits - m)
    lse = jnp.log(jnp.sum(ex, axis=1, keepdims=True))
    o_ref[...] = logits - m - lse


def _final(o0, o1, b2r):
    return pl.pallas_call(
        _final_body,
        grid=(NBLK,),
        in_specs=[
            pl.BlockSpec((BLK, CLS), lambda i: (i, 0)),
            pl.BlockSpec((BLK, CLS), lambda i: (i, 0)),
            pl.BlockSpec((1, CLS), lambda i: (0, 0)),
        ],
        out_specs=pl.BlockSpec((BLK, CLS), lambda i: (i, 0)),
        out_shape=jax.ShapeDtypeStruct((NPAD, CLS), jnp.float32),
    )(o0, o1, b2r)


def kernel(x, edge_index, W1, att_src1, att_dst1, b1, W2, att_src2, att_dst2, b2):
    xp = jnp.concatenate(
        [x.astype(jnp.float32), jnp.zeros((NPAD - N, F), jnp.float32)])
    loop = jnp.arange(N, dtype=jnp.int32)
    pad = EPAD - E
    src = jnp.concatenate([edge_index[0].astype(jnp.int32), loop,
                           jnp.zeros((pad,), jnp.int32)])
    dst = jnp.concatenate([edge_index[1].astype(jnp.int32), loop,
                           jnp.full((pad,), N, jnp.int32)])

    h3, asrc1, adst1 = _mm1(xp, W1, att_src1, att_dst1)
    s1, d10, d11 = _edge_softmax(asrc1, adst1, src, dst)
    al1 = _alpha1(s1, dst, d10, d11)
    out1f = _msg1(h3.reshape(16 * NPAD, 128), src, dst, al1)

    ev = 2 * jnp.arange(16, dtype=jnp.int32)
    perm32 = jnp.concatenate([ev, ev + 1])
    perm128 = jnp.concatenate([b * 32 + perm32 for b in range(4)])
    h2, a2s, a2d = _mm2(out1f.reshape(16, NPAD, 128),
                        W2.reshape(16, 128, CLS)[:, perm128, :],
                        b1.reshape(16, 128)[:, perm128],
                        att_src2, att_dst2)
    s2, d20, d21 = _edge_softmax(a2s, a2d, src, dst)
    o20, o21 = _msg2(h2, src, dst, s2, d20, d21)

    out = _final(o20, o21, b2.reshape(1, CLS))
    return out[:N]


# speedup vs baseline: 10.3178x; 1.0006x over previous
import functools

import jax
import jax.numpy as jnp
from jax import lax
from jax.experimental import pallas as pl
from jax.experimental.pallas import tpu as pltpu
from jax.experimental.pallas import tpu_sc as plsc

N = 10000
F = 256
HID = 256
H = 8
CLS = 16
E0 = 160000

NC, NS, L = 2, 16, 16
NW = NC * NS

NPAD = 10240
BLK = 320
NBLK = NPAD // BLK
RPS = NPAD // NS

E = E0 + N
EPT = 5376
EPAD = EPT * NW
CH = 128
NCH_W = EPT // CH
EPT_S = EPAD // NS
NCH_S = EPT_S // CH

_SC_PARAMS = pltpu.CompilerParams(needs_layout_passes=False,
                                  use_tc_tiling_on_sc=False)
_MESH = plsc.VectorSubcoreMesh(core_axis_name="c", subcore_axis_name="s")


def _mm1_body(x_ref, w_ref, asw_ref, adw_ref, h3_ref, asrc_ref, adst_ref):
    hb = jnp.dot(x_ref[...], w_ref[...], preferred_element_type=jnp.float32)
    for s in range(16):
        h3_ref[s, :, :] = hb[:, s * 128:(s + 1) * 128].astype(jnp.bfloat16)
    for h in range(H):
        seg = hb[:, h * HID:(h + 1) * HID]
        asrc_ref[:, h:h + 1] = jnp.sum(seg * asw_ref[h:h + 1, :], axis=1,
                                       keepdims=True)
        adst_ref[:, h:h + 1] = jnp.sum(seg * adw_ref[h:h + 1, :], axis=1,
                                       keepdims=True)
    asrc_ref[:, H:] = jnp.zeros((BLK, 16 - H), jnp.float32)
    adst_ref[:, H:] = jnp.zeros((BLK, 16 - H), jnp.float32)


def _mm1(xp, W1, att_src1, att_dst1):
    return pl.pallas_call(
        _mm1_body,
        grid=(NBLK,),
        in_specs=[
            pl.BlockSpec((BLK, F), lambda i: (i, 0)),
            pl.BlockSpec((F, H * HID), lambda i: (0, 0)),
            pl.BlockSpec((H, HID), lambda i: (0, 0)),
            pl.BlockSpec((H, HID), lambda i: (0, 0)),
        ],
        out_specs=[
            pl.BlockSpec((16, BLK, 128), lambda i: (0, i, 0)),
            pl.BlockSpec((BLK, 16), lambda i: (i, 0)),
            pl.BlockSpec((BLK, 16), lambda i: (i, 0)),
        ],
        out_shape=[
            jax.ShapeDtypeStruct((16, NPAD, 128), jnp.bfloat16),
            jax.ShapeDtypeStruct((NPAD, 16), jnp.float32),
            jax.ShapeDtypeStruct((NPAD, 16), jnp.float32),
        ],
    )(xp, W1, att_src1, att_dst1)


@functools.partial(
    pl.kernel,
    out_type=[
        jax.ShapeDtypeStruct((EPAD, 16), jnp.float32),
        jax.ShapeDtypeStruct((NPAD, 16), jnp.float32),
        jax.ShapeDtypeStruct((NPAD, 16), jnp.float32),
    ],
    mesh=_MESH,
    compiler_params=_SC_PARAMS,
    scratch_types=[
        pltpu.VMEM((CH,), jnp.int32),
        pltpu.VMEM((CH,), jnp.int32),
        pltpu.VMEM((CH, 16), jnp.float32),
        pltpu.VMEM((CH, 16), jnp.float32),
        pltpu.VMEM((CH, 16), jnp.float32),
        pltpu.VMEM((RPS, 16), jnp.float32),
        pltpu.VMEM_SHARED((NPAD, 16), jnp.float32),
        pltpu.SemaphoreType.DMA,
        pltpu.SemaphoreType.DMA,
    ],
)
def _edge_softmax(asrc_hbm, adst_hbm, src_hbm, dst_hbm,
                  s_out, d0_out, d1_out,
                  src_v, dst_v, asr, adr, s_blk, zbuf, den_sh, sem1, sem2):
    c = lax.axis_index("c")
    s = lax.axis_index("s")
    wid = c * NS + s

    def zrow(i, carry):
        zbuf[i, :] = jnp.zeros((L,), jnp.float32)
        return carry
    lax.fori_loop(0, RPS, zrow, 0)
    pltpu.sync_copy(zbuf, den_sh.at[pl.ds(s * RPS, RPS)])
    plsc.subcore_barrier()

    def chunk(i, carry):
        base = wid * EPT + i * CH
        pltpu.sync_copy(src_hbm.at[pl.ds(base, CH)], src_v)
        pltpu.sync_copy(dst_hbm.at[pl.ds(base, CH)], dst_v)
        cp1 = pltpu.async_copy(asrc_hbm.at[src_v], asr, sem1)
        cp2 = pltpu.async_copy(adst_hbm.at[dst_v], adr, sem2)
        cp1.wait()
        cp2.wait()

        def row(j, carry2):
            e = asr[j, :] + adr[j, :]
            e = jnp.maximum(e, 0.2 * e)
            s_blk[j, :] = jnp.exp(e)
            return carry2
        lax.fori_loop(0, CH, row, 0)

        pltpu.sync_copy(s_blk, s_out.at[pl.ds(base, CH)])
        pltpu.sync_copy(s_blk, den_sh.at[dst_v], add=True)
        return carry
    lax.fori_loop(0, NCH_W, chunk, 0)
    plsc.subcore_barrier()

    @pl.when(c == 0)
    def _():
        pltpu.sync_copy(den_sh.at[pl.ds(s * RPS, RPS)],
                        d0_out.at[pl.ds(s * RPS, RPS)])

    @pl.when(c == 1)
    def _():
        pltpu.sync_copy(den_sh.at[pl.ds(s * RPS, RPS)],
                        d1_out.at[pl.ds(s * RPS, RPS)])


@functools.partial(
    pl.kernel,
    out_type=jax.ShapeDtypeStruct((8 * EPAD,), jnp.float32),
    mesh=_MESH,
    compiler_params=_SC_PARAMS,
    scratch_types=[
        pltpu.VMEM((CH,), jnp.int32),
        pltpu.VMEM((CH, 16), jnp.float32),
        pltpu.VMEM((CH, 16), jnp.float32),
        pltpu.VMEM((CH, 16), jnp.float32),
        pltpu.VMEM((8, CH), jnp.float32),
        pltpu.SemaphoreType.DMA,
        pltpu.SemaphoreType.DMA,
    ],
)
def _alpha1(s_hbm, dst_hbm, d0_hbm, d1_hbm, al_out,
            dst_v, s_blk, dr0, dr1, al8, sem1, sem2):
    c = lax.axis_index("c")
    s = lax.axis_index("s")
    wid = c * NS + s

    def chunk(i, carry):
        base = wid * EPT + i * CH
        pltpu.sync_copy(dst_hbm.at[pl.ds(base, CH)], dst_v)
        pltpu.sync_copy(s_hbm.at[pl.ds(base, CH)], s_blk)
        cp1 = pltpu.async_copy(d0_hbm.at[dst_v], dr0, sem1)
        cp2 = pltpu.async_copy(d1_hbm.at[dst_v], dr1, sem2)
        cp1.wait()
        cp2.wait()
        for h in range(H):
            hv = jnp.full((L,), h, jnp.int32)
            for g in range(CH // L):
                ev = lax.iota(jnp.int32, L) + g * L
                sc = plsc.load_gather(s_blk, [ev, hv])
                dc0 = plsc.load_gather(dr0, [ev, hv])
                dc1 = plsc.load_gather(dr1, [ev, hv])
                al8[h, pl.ds(g * L, L)] = sc / (dc0 + dc1)
        for h in range(H):
            pltpu.sync_copy(al8.at[h], al_out.at[pl.ds(h * EPAD + base, CH)])
        return carry
    lax.fori_loop(0, NCH_W, chunk, 0)


@functools.partial(
    pl.kernel,
    out_type=jax.ShapeDtypeStruct((16 * NPAD, 128), jnp.float32),
    mesh=_MESH,
    compiler_params=_SC_PARAMS,
    scratch_types=[
        pltpu.VMEM((CH,), jnp.int32),
        pltpu.VMEM((CH,), jnp.int32),
        pltpu.VMEM((CH,), jnp.int32),
        pltpu.VMEM((CH,), jnp.int32),
        pltpu.VMEM((CH,), jnp.int32),
        pltpu.VMEM((CH,), jnp.int32),
        pltpu.VMEM((CH,), jnp.float32),
        pltpu.VMEM((CH,), jnp.float32),
        pltpu.VMEM((CH, 128), jnp.bfloat16),
        pltpu.VMEM((CH, 128), jnp.bfloat16),
        pltpu.VMEM((CH, 128), jnp.float32),
        pltpu.VMEM((16, 128), jnp.float32),
        pltpu.VMEM_SHARED((NPAD, 128), jnp.float32),
        pltpu.SemaphoreType.DMA,
        pltpu.SemaphoreType.DMA,
    ],
)
def _msg1(h3_hbm, src_hbm, dst_hbm, al_hbm, out_hbm,
          src_a, src_b, dst_a, dst_b, gidx_a, gidx_b, al_a, al_b,
          rows_a, rows_b, rows32, zbuf, acc_sh, sem_ga, sem_gb):
    c = lax.axis_index("c")
    s = lax.axis_index("s")

    def zrow(i, carry):
        for k in range(128 // L):
            zbuf[i, pl.ds(k * L, L)] = jnp.zeros((L,), jnp.float32)
        return carry
    lax.fori_loop(0, 16, zrow, 0)

    def _load_meta(i, slab, head, src_v, dst_v, gidx, al):
        base = s * EPT_S + i * CH
        pltpu.sync_copy(src_hbm.at[pl.ds(base, CH)], src_v)
        for g in range(CH // L):
            gidx[pl.ds(g * L, L)] = src_v[pl.ds(g * L, L)] + slab * NPAD
        pltpu.sync_copy(dst_hbm.at[pl.ds(base, CH)], dst_v)
        pltpu.sync_copy(al_hbm.at[pl.ds(head * EPAD + base, CH)], al)

    def _scale(rows, al):
        @plsc.parallel_loop(0, CH, 1, unroll=8)
        def _(e):
            av = plsc.load_gather(al, [jnp.full((L,), e, jnp.int32)])
            for k in range(128 // 32):
                x = rows[e, pl.ds(k * 32, 32)]
                u0, u1 = plsc.unpack(x, format=plsc.PackFormat.INTERLEAVED)
                rows32[e, pl.ds(k * 32, L)] = u0 * av
                rows32[e, pl.ds(k * 32 + L, L)] = u1 * av

    def slab_loop(j, carry):
        slab = c * 8 + j
        head = slab // 2

        def zcp(k, carry2):
            pltpu.sync_copy(zbuf, acc_sh.at[pl.ds(s * RPS + k * 16, 16)])
            return carry2
        lax.fori_loop(0, RPS // 16, zcp, 0)
        plsc.subcore_barrier()

        _load_meta(0, slab, head, src_a, dst_a, gidx_a, al_a)
        cp_a0 = pltpu.async_copy(h3_hbm.at[gidx_a], rows_a, sem_ga)
        cp_a0.wait()

        def pair(p, carry2):
            ia = 2 * p
            ib = 2 * p + 1
            _load_meta(ib, slab, head, src_b, dst_b, gidx_b, al_b)
            cp_gb = pltpu.async_copy(h3_hbm.at[gidx_b], rows_b, sem_gb)
            _scale(rows_a, al_a)
            pltpu.sync_copy(rows32, acc_sh.at[dst_a], add=True)
            cp_gb.wait()

            @pl.when(ia + 2 < NCH_S)
            def _():
                _load_meta(ia + 2, slab, head, src_a, dst_a, gidx_a, al_a)
                cp_ga = pltpu.async_copy(h3_hbm.at[gidx_a], rows_a, sem_ga)
                _scale(rows_b, al_b)
                pltpu.sync_copy(rows32, acc_sh.at[dst_b], add=True)
                cp_ga.wait()

            @pl.when(ia + 2 >= NCH_S)
            def _():
                _scale(rows_b, al_b)
                pltpu.sync_copy(rows32, acc_sh.at[dst_b], add=True)
            return carry2
        lax.fori_loop(0, NCH_S // 2, pair, 0)
        plsc.subcore_barrier()

        pltpu.sync_copy(acc_sh.at[pl.ds(s * RPS, RPS)],
                        out_hbm.at[pl.ds(slab * NPAD + s * RPS, RPS)])
        return carry
    lax.fori_loop(0, 8, slab_loop, 0)


def _mm2_body(o1_ref, w2_ref, b1_ref, asw_ref, adw_ref,
              h2_ref, a2s_ref, a2d_ref):
    acc = jnp.zeros((BLK, CLS), jnp.float32)
    for sl in range(16):
        hb = o1_ref[sl] + b1_ref[sl:sl + 1, :]
        hb = jnp.where(hb > 0, hb, jnp.exp(jnp.minimum(hb, 0.0)) - 1.0)
        acc = acc + jnp.dot(hb, w2_ref[sl], preferred_element_type=jnp.float32)
    h2_ref[...] = acc
    a2s = jnp.sum(acc * asw_ref[...], axis=1, keepdims=True)
    a2d = jnp.sum(acc * adw_ref[...], axis=1, keepdims=True)
    a2s_ref[...] = jnp.broadcast_to(a2s, (BLK, 16))
    a2d_ref[...] = jnp.broadcast_to(a2d, (BLK, 16))


def _mm2(out1, W2r, b1r, att_src2, att_dst2):
    return pl.pallas_call(
        _mm2_body,
        grid=(NBLK,),
        in_specs=[
            pl.BlockSpec((16, BLK, 128), lambda i: (0, i, 0)),
            pl.BlockSpec((16, 128, CLS), lambda i: (0, 0, 0)),
            pl.BlockSpec((16, 128), lambda i: (0, 0)),
            pl.BlockSpec((1, CLS), lambda i: (0, 0)),
            pl.BlockSpec((1, CLS), lambda i: (0, 0)),
        ],
        out_specs=[
            pl.BlockSpec((BLK, CLS), lambda i: (i, 0)),
            pl.BlockSpec((BLK, 16), lambda i: (i, 0)),
            pl.BlockSpec((BLK, 16), lambda i: (i, 0)),
        ],
        out_shape=[
            jax.ShapeDtypeStruct((NPAD, CLS), jnp.float32),
            jax.ShapeDtypeStruct((NPAD, 16), jnp.float32),
            jax.ShapeDtypeStruct((NPAD, 16), jnp.float32),
        ],
    )(out1, W2r, b1r, att_src2, att_dst2)


@functools.partial(
    pl.kernel,
    out_type=[
        jax.ShapeDtypeStruct((NPAD, 16), jnp.float32),
        jax.ShapeDtypeStruct((NPAD, 16), jnp.float32),
    ],
    mesh=_MESH,
    compiler_params=_SC_PARAMS,
    scratch_types=[
        pltpu.VMEM((CH,), jnp.int32),
        pltpu.VMEM((CH,), jnp.int32),
        pltpu.VMEM((CH, 16), jnp.float32),
        pltpu.VMEM((CH, 16), jnp.float32),
        pltpu.VMEM((CH, 16), jnp.float32),
        pltpu.VMEM((CH, 16), jnp.float32),
        pltpu.VMEM((RPS, 16), jnp.float32),
        pltpu.VMEM_SHARED((NPAD, 16), jnp.float32),
        pltpu.SemaphoreType.DMA,
        pltpu.SemaphoreType.DMA,
        pltpu.SemaphoreType.DMA,
    ],
)
def _msg2(h2_hbm, src_hbm, dst_hbm, s_hbm, d0_hbm, d1_hbm,
          o0_out, o1_out,
          src_v, dst_v, s_blk, dr0, dr1, rows, zbuf, acc_sh,
          sem1, sem2, sem3):
    c = lax.axis_index("c")
    s = lax.axis_index("s")
    wid = c * NS + s

    def zrow(i, carry):
        zbuf[i, :] = jnp.zeros((L,), jnp.float32)
        return carry
    lax.fori_loop(0, RPS, zrow, 0)
    pltpu.sync_copy(zbuf, acc_sh.at[pl.ds(s * RPS, RPS)])
    plsc.subcore_barrier()

    def chunk(i, carry):
        base = wid * EPT + i * CH
        pltpu.sync_copy(src_hbm.at[pl.ds(base, CH)], src_v)
        pltpu.sync_copy(dst_hbm.at[pl.ds(base, CH)], dst_v)
        pltpu.sync_copy(s_hbm.at[pl.ds(base, CH)], s_blk)
        cp1 = pltpu.async_copy(d0_hbm.at[dst_v], dr0, sem1)
        cp2 = pltpu.async_copy(d1_hbm.at[dst_v], dr1, sem2)
        cp3 = pltpu.async_copy(h2_hbm.at[src_v], rows, sem3)
        cp1.wait()
        cp2.wait()
        cp3.wait()

        def row(e, carry2):
            alpha = s_blk[e, :] / (dr0[e, :] + dr1[e, :])
            rows[e, :] = rows[e, :] * alpha
            return carry2
        lax.fori_loop(0, CH, row, 0)

        pltpu.sync_copy(rows, acc_sh.at[dst_v], add=True)
        return carry
    lax.fori_loop(0, NCH_W, chunk, 0)
    plsc.subcore_barrier()

    @pl.when(c == 0)
    def _():
        pltpu.sync_copy(acc_sh.at[pl.ds(s * RPS, RPS)],
                        o0_out.at[pl.ds(s * RPS, RPS)])

    @pl.when(c == 1)
    def _():
        pltpu.sync_copy(acc_sh.at[pl.ds(s * RPS, RPS)],
                        o1_out.at[pl.ds(s * RPS, RPS)])


def _final_body(p0_ref, p1_ref, b2_ref, o_ref):
    logits = p0_ref[...] + p1_ref[...] + b2_ref[...]
    m = jnp.max(logits, axis=1, keepdims=True)
    ex = jnp.exp(logits - m)
    lse = jnp.log(jnp.sum(ex, axis=1, keepdims=True))
    o_ref[...] = logits - m - lse


def _final(o0, o1, b2r):
    return pl.pallas_call(
        _final_body,
        grid=(NBLK,),
        in_specs=[
            pl.BlockSpec((BLK, CLS), lambda i: (i, 0)),
            pl.BlockSpec((BLK, CLS), lambda i: (i, 0)),
            pl.BlockSpec((1, CLS), lambda i: (0, 0)),
        ],
        out_specs=pl.BlockSpec((BLK, CLS), lambda i: (i, 0)),
        out_shape=jax.ShapeDtypeStruct((NPAD, CLS), jnp.float32),
    )(o0, o1, b2r)


def kernel(x, edge_index, W1, att_src1, att_dst1, b1, W2, att_src2, att_dst2, b2):
    xp = jnp.concatenate(
        [x.astype(jnp.float32), jnp.zeros((NPAD - N, F), jnp.float32)])
    loop = jnp.arange(N, dtype=jnp.int32)
    pad = EPAD - E
    src = jnp.concatenate([edge_index[0].astype(jnp.int32), loop,
                           jnp.zeros((pad,), jnp.int32)])
    dst = jnp.concatenate([edge_index[1].astype(jnp.int32), loop,
                           jnp.full((pad,), N, jnp.int32)])

    h3, asrc1, adst1 = _mm1(xp, W1, att_src1, att_dst1)
    s1, d10, d11 = _edge_softmax(asrc1, adst1, src, dst)
    al1 = _alpha1(s1, dst, d10, d11)
    out1f = _msg1(h3.reshape(16 * NPAD, 128), src, dst, al1)

    ev = 2 * jnp.arange(16, dtype=jnp.int32)
    perm32 = jnp.concatenate([ev, ev + 1])
    perm128 = jnp.concatenate([b * 32 + perm32 for b in range(4)])
    h2, a2s, a2d = _mm2(out1f.reshape(16, NPAD, 128),
                        W2.reshape(16, 128, CLS)[:, perm128, :],
                        b1.reshape(16, 128)[:, perm128],
                        att_src2, att_dst2)
    s2, d20, d21 = _edge_softmax(a2s, a2d, src, dst)
    o20, o21 = _msg2(h2, src, dst, s2, d20, d21)

    out = _final(o20, o21, b2.reshape(1, CLS))
    return out[:N]


# speedup vs baseline: 10.3198x; 1.0002x over previous
import functools

import jax
import jax.numpy as jnp
from jax import lax
from jax.experimental import pallas as pl
from jax.experimental.pallas import tpu as pltpu
from jax.experimental.pallas import tpu_sc as plsc

N = 10000
F = 256
HID = 256
H = 8
CLS = 16
E0 = 160000

NC, NS, L = 2, 16, 16
NW = NC * NS

NPAD = 10240
BLK = 320
NBLK = NPAD // BLK
RPS = NPAD // NS

E = E0 + N
EPT = 5376
EPAD = EPT * NW
CH = 128
NCH_W = EPT // CH
EPT_S = EPAD // NS
NCH_S = EPT_S // CH

_SC_PARAMS = pltpu.CompilerParams(needs_layout_passes=False,
                                  use_tc_tiling_on_sc=False)
_MESH = plsc.VectorSubcoreMesh(core_axis_name="c", subcore_axis_name="s")


def _mm1_body(x_ref, w_ref, asw_ref, adw_ref, h3_ref, asrc_ref, adst_ref):
    hb = jnp.dot(x_ref[...], w_ref[...], preferred_element_type=jnp.float32)
    for s in range(16):
        h3_ref[s, :, :] = hb[:, s * 128:(s + 1) * 128].astype(jnp.bfloat16)
    for h in range(H):
        seg = hb[:, h * HID:(h + 1) * HID]
        asrc_ref[:, h:h + 1] = jnp.sum(seg * asw_ref[h:h + 1, :], axis=1,
                                       keepdims=True)
        adst_ref[:, h:h + 1] = jnp.sum(seg * adw_ref[h:h + 1, :], axis=1,
                                       keepdims=True)
    asrc_ref[:, H:] = jnp.zeros((BLK, 16 - H), jnp.float32)
    adst_ref[:, H:] = jnp.zeros((BLK, 16 - H), jnp.float32)


def _mm1(xp, W1, att_src1, att_dst1):
    return pl.pallas_call(
        _mm1_body,
        grid=(NBLK,),
        in_specs=[
            pl.BlockSpec((BLK, F), lambda i: (i, 0)),
            pl.BlockSpec((F, H * HID), lambda i: (0, 0)),
            pl.BlockSpec((H, HID), lambda i: (0, 0)),
            pl.BlockSpec((H, HID), lambda i: (0, 0)),
        ],
        out_specs=[
            pl.BlockSpec((16, BLK, 128), lambda i: (0, i, 0)),
            pl.BlockSpec((BLK, 16), lambda i: (i, 0)),
            pl.BlockSpec((BLK, 16), lambda i: (i, 0)),
        ],
        out_shape=[
            jax.ShapeDtypeStruct((16, NPAD, 128), jnp.bfloat16),
            jax.ShapeDtypeStruct((NPAD, 16), jnp.float32),
            jax.ShapeDtypeStruct((NPAD, 16), jnp.float32),
        ],
    )(xp, W1, att_src1, att_dst1)


@functools.partial(
    pl.kernel,
    out_type=[
        jax.ShapeDtypeStruct((EPAD, 16), jnp.float32),
        jax.ShapeDtypeStruct((NPAD, 16), jnp.float32),
        jax.ShapeDtypeStruct((NPAD, 16), jnp.float32),
    ],
    mesh=_MESH,
    compiler_params=_SC_PARAMS,
    scratch_types=[
        pltpu.VMEM((CH,), jnp.int32),
        pltpu.VMEM((CH,), jnp.int32),
        pltpu.VMEM((CH, 16), jnp.float32),
        pltpu.VMEM((CH, 16), jnp.float32),
        pltpu.VMEM((CH, 16), jnp.float32),
        pltpu.VMEM((RPS, 16), jnp.float32),
        pltpu.VMEM_SHARED((NPAD, 16), jnp.float32),
        pltpu.SemaphoreType.DMA,
        pltpu.SemaphoreType.DMA,
    ],
)
def _edge_softmax(asrc_hbm, adst_hbm, src_hbm, dst_hbm,
                  s_out, d0_out, d1_out,
                  src_v, dst_v, asr, adr, s_blk, zbuf, den_sh, sem1, sem2):
    c = lax.axis_index("c")
    s = lax.axis_index("s")
    wid = c * NS + s

    def zrow(i, carry):
        zbuf[i, :] = jnp.zeros((L,), jnp.float32)
        return carry
    lax.fori_loop(0, RPS, zrow, 0)
    pltpu.sync_copy(zbuf, den_sh.at[pl.ds(s * RPS, RPS)])
    plsc.subcore_barrier()

    def chunk(i, carry):
        base = wid * EPT + i * CH
        pltpu.sync_copy(src_hbm.at[pl.ds(base, CH)], src_v)
        pltpu.sync_copy(dst_hbm.at[pl.ds(base, CH)], dst_v)
        cp1 = pltpu.async_copy(asrc_hbm.at[src_v], asr, sem1)
        cp2 = pltpu.async_copy(adst_hbm.at[dst_v], adr, sem2)
        cp1.wait()
        cp2.wait()

        def row(j, carry2):
            e = asr[j, :] + adr[j, :]
            e = jnp.maximum(e, 0.2 * e)
            s_blk[j, :] = jnp.exp(e)
            return carry2
        lax.fori_loop(0, CH, row, 0)

        pltpu.sync_copy(s_blk, s_out.at[pl.ds(base, CH)])
        pltpu.sync_copy(s_blk, den_sh.at[dst_v], add=True)
        return carry
    lax.fori_loop(0, NCH_W, chunk, 0)
    plsc.subcore_barrier()

    @pl.when(c == 0)
    def _():
        pltpu.sync_copy(den_sh.at[pl.ds(s * RPS, RPS)],
                        d0_out.at[pl.ds(s * RPS, RPS)])

    @pl.when(c == 1)
    def _():
        pltpu.sync_copy(den_sh.at[pl.ds(s * RPS, RPS)],
                        d1_out.at[pl.ds(s * RPS, RPS)])


@functools.partial(
    pl.kernel,
    out_type=jax.ShapeDtypeStruct((8 * EPAD,), jnp.float32),
    mesh=_MESH,
    compiler_params=_SC_PARAMS,
    scratch_types=[
        pltpu.VMEM((CH,), jnp.int32),
        pltpu.VMEM((CH, 16), jnp.float32),
        pltpu.VMEM((CH, 16), jnp.float32),
        pltpu.VMEM((CH, 16), jnp.float32),
        pltpu.VMEM((8, CH), jnp.float32),
        pltpu.SemaphoreType.DMA,
        pltpu.SemaphoreType.DMA,
    ],
)
def _alpha1(s_hbm, dst_hbm, d0_hbm, d1_hbm, al_out,
            dst_v, s_blk, dr0, dr1, al8, sem1, sem2):
    c = lax.axis_index("c")
    s = lax.axis_index("s")
    wid = c * NS + s

    def chunk(i, carry):
        base = wid * EPT + i * CH
        pltpu.sync_copy(dst_hbm.at[pl.ds(base, CH)], dst_v)
        pltpu.sync_copy(s_hbm.at[pl.ds(base, CH)], s_blk)
        cp1 = pltpu.async_copy(d0_hbm.at[dst_v], dr0, sem1)
        cp2 = pltpu.async_copy(d1_hbm.at[dst_v], dr1, sem2)
        cp1.wait()
        cp2.wait()
        for h in range(H):
            hv = jnp.full((L,), h, jnp.int32)
            for g in range(CH // L):
                ev = lax.iota(jnp.int32, L) + g * L
                sc = plsc.load_gather(s_blk, [ev, hv])
                dc0 = plsc.load_gather(dr0, [ev, hv])
                dc1 = plsc.load_gather(dr1, [ev, hv])
                al8[h, pl.ds(g * L, L)] = sc / (dc0 + dc1)
        for h in range(H):
            pltpu.sync_copy(al8.at[h], al_out.at[pl.ds(h * EPAD + base, CH)])
        return carry
    lax.fori_loop(0, NCH_W, chunk, 0)


@functools.partial(
    pl.kernel,
    out_type=jax.ShapeDtypeStruct((16 * NPAD, 128), jnp.float32),
    mesh=_MESH,
    compiler_params=_SC_PARAMS,
    scratch_types=[
        pltpu.VMEM((CH,), jnp.int32),
        pltpu.VMEM((CH,), jnp.int32),
        pltpu.VMEM((CH,), jnp.int32),
        pltpu.VMEM((CH,), jnp.int32),
        pltpu.VMEM((CH,), jnp.int32),
        pltpu.VMEM((CH,), jnp.int32),
        pltpu.VMEM((CH,), jnp.float32),
        pltpu.VMEM((CH,), jnp.float32),
        pltpu.VMEM((CH, 128), jnp.bfloat16),
        pltpu.VMEM((CH, 128), jnp.bfloat16),
        pltpu.VMEM((CH, 128), jnp.float32),
        pltpu.VMEM((16, 128), jnp.float32),
        pltpu.VMEM_SHARED((NPAD, 128), jnp.float32),
        pltpu.SemaphoreType.DMA,
        pltpu.SemaphoreType.DMA,
    ],
)
def _msg1(h3_hbm, src_hbm, dst_hbm, al_hbm, out_hbm,
          src_a, src_b, dst_a, dst_b, gidx_a, gidx_b, al_a, al_b,
          rows_a, rows_b, rows32, zbuf, acc_sh, sem_ga, sem_gb):
    c = lax.axis_index("c")
    s = lax.axis_index("s")

    def zrow(i, carry):
        for k in range(128 // L):
            zbuf[i, pl.ds(k * L, L)] = jnp.zeros((L,), jnp.float32)
        return carry
    lax.fori_loop(0, 16, zrow, 0)

    def _load_meta(i, slab, head, src_v, dst_v, gidx, al):
        base = s * EPT_S + i * CH
        pltpu.sync_copy(src_hbm.at[pl.ds(base, CH)], src_v)
        for g in range(CH // L):
            gidx[pl.ds(g * L, L)] = src_v[pl.ds(g * L, L)] + slab * NPAD
        pltpu.sync_copy(dst_hbm.at[pl.ds(base, CH)], dst_v)
        pltpu.sync_copy(al_hbm.at[pl.ds(head * EPAD + base, CH)], al)

    def _scale(rows, al):
        @plsc.parallel_loop(0, CH, 1, unroll=4)
        def _(e):
            av = plsc.load_gather(al, [jnp.full((L,), e, jnp.int32)])
            for k in range(128 // 32):
                x = rows[e, pl.ds(k * 32, 32)]
                u0, u1 = plsc.unpack(x, format=plsc.PackFormat.INTERLEAVED)
                rows32[e, pl.ds(k * 32, L)] = u0 * av
                rows32[e, pl.ds(k * 32 + L, L)] = u1 * av

    def slab_loop(j, carry):
        slab = c * 8 + j
        head = slab // 2

        def zcp(k, carry2):
            pltpu.sync_copy(zbuf, acc_sh.at[pl.ds(s * RPS + k * 16, 16)])
            return carry2
        lax.fori_loop(0, RPS // 16, zcp, 0)
        plsc.subcore_barrier()

        _load_meta(0, slab, head, src_a, dst_a, gidx_a, al_a)
        cp_a0 = pltpu.async_copy(h3_hbm.at[gidx_a], rows_a, sem_ga)
        cp_a0.wait()

        def pair(p, carry2):
            ia = 2 * p
            ib = 2 * p + 1
            _load_meta(ib, slab, head, src_b, dst_b, gidx_b, al_b)
            cp_gb = pltpu.async_copy(h3_hbm.at[gidx_b], rows_b, sem_gb)
            _scale(rows_a, al_a)
            pltpu.sync_copy(rows32, acc_sh.at[dst_a], add=True)
            cp_gb.wait()

            @pl.when(ia + 2 < NCH_S)
            def _():
                _load_meta(ia + 2, slab, head, src_a, dst_a, gidx_a, al_a)
                cp_ga = pltpu.async_copy(h3_hbm.at[gidx_a], rows_a, sem_ga)
                _scale(rows_b, al_b)
                pltpu.sync_copy(rows32, acc_sh.at[dst_b], add=True)
                cp_ga.wait()

            @pl.when(ia + 2 >= NCH_S)
            def _():
                _scale(rows_b, al_b)
                pltpu.sync_copy(rows32, acc_sh.at[dst_b], add=True)
            return carry2
        lax.fori_loop(0, NCH_S // 2, pair, 0)
        plsc.subcore_barrier()

        pltpu.sync_copy(acc_sh.at[pl.ds(s * RPS, RPS)],
                        out_hbm.at[pl.ds(slab * NPAD + s * RPS, RPS)])
        return carry
    lax.fori_loop(0, 8, slab_loop, 0)


def _mm2_body(o1_ref, w2_ref, b1_ref, asw_ref, adw_ref,
              h2_ref, a2s_ref, a2d_ref):
    acc = jnp.zeros((BLK, CLS), jnp.float32)
    for sl in range(16):
        hb = o1_ref[sl] + b1_ref[sl:sl + 1, :]
        hb = jnp.where(hb > 0, hb, jnp.exp(jnp.minimum(hb, 0.0)) - 1.0)
        acc = acc + jnp.dot(hb, w2_ref[sl], preferred_element_type=jnp.float32)
    h2_ref[...] = acc
    a2s = jnp.sum(acc * asw_ref[...], axis=1, keepdims=True)
    a2d = jnp.sum(acc * adw_ref[...], axis=1, keepdims=True)
    a2s_ref[...] = jnp.broadcast_to(a2s, (BLK, 16))
    a2d_ref[...] = jnp.broadcast_to(a2d, (BLK, 16))


def _mm2(out1, W2r, b1r, att_src2, att_dst2):
    return pl.pallas_call(
        _mm2_body,
        grid=(NBLK,),
        in_specs=[
            pl.BlockSpec((16, BLK, 128), lambda i: (0, i, 0)),
            pl.BlockSpec((16, 128, CLS), lambda i: (0, 0, 0)),
            pl.BlockSpec((16, 128), lambda i: (0, 0)),
            pl.BlockSpec((1, CLS), lambda i: (0, 0)),
            pl.BlockSpec((1, CLS), lambda i: (0, 0)),
        ],
        out_specs=[
            pl.BlockSpec((BLK, CLS), lambda i: (i, 0)),
            pl.BlockSpec((BLK, 16), lambda i: (i, 0)),
            pl.BlockSpec((BLK, 16), lambda i: (i, 0)),
        ],
        out_shape=[
            jax.ShapeDtypeStruct((NPAD, CLS), jnp.float32),
            jax.ShapeDtypeStruct((NPAD, 16), jnp.float32),
            jax.ShapeDtypeStruct((NPAD, 16), jnp.float32),
        ],
    )(out1, W2r, b1r, att_src2, att_dst2)


@functools.partial(
    pl.kernel,
    out_type=[
        jax.ShapeDtypeStruct((NPAD, 16), jnp.float32),
        jax.ShapeDtypeStruct((NPAD, 16), jnp.float32),
    ],
    mesh=_MESH,
    compiler_params=_SC_PARAMS,
    scratch_types=[
        pltpu.VMEM((CH,), jnp.int32),
        pltpu.VMEM((CH,), jnp.int32),
        pltpu.VMEM((CH, 16), jnp.float32),
        pltpu.VMEM((CH, 16), jnp.float32),
        pltpu.VMEM((CH, 16), jnp.float32),
        pltpu.VMEM((CH, 16), jnp.float32),
        pltpu.VMEM((RPS, 16), jnp.float32),
        pltpu.VMEM_SHARED((NPAD, 16), jnp.float32),
        pltpu.SemaphoreType.DMA,
        pltpu.SemaphoreType.DMA,
        pltpu.SemaphoreType.DMA,
    ],
)
def _msg2(h2_hbm, src_hbm, dst_hbm, s_hbm, d0_hbm, d1_hbm,
          o0_out, o1_out,
          src_v, dst_v, s_blk, dr0, dr1, rows, zbuf, acc_sh,
          sem1, sem2, sem3):
    c = lax.axis_index("c")
    s = lax.axis_index("s")
    wid = c * NS + s

    def zrow(i, carry):
        zbuf[i, :] = jnp.zeros((L,), jnp.float32)
        return carry
    lax.fori_loop(0, RPS, zrow, 0)
    pltpu.sync_copy(zbuf, acc_sh.at[pl.ds(s * RPS, RPS)])
    plsc.subcore_barrier()

    def chunk(i, carry):
        base = wid * EPT + i * CH
        pltpu.sync_copy(src_hbm.at[pl.ds(base, CH)], src_v)
        pltpu.sync_copy(dst_hbm.at[pl.ds(base, CH)], dst_v)
        pltpu.sync_copy(s_hbm.at[pl.ds(base, CH)], s_blk)
        cp1 = pltpu.async_copy(d0_hbm.at[dst_v], dr0, sem1)
        cp2 = pltpu.async_copy(d1_hbm.at[dst_v], dr1, sem2)
        cp3 = pltpu.async_copy(h2_hbm.at[src_v], rows, sem3)
        cp1.wait()
        cp2.wait()
        cp3.wait()

        def row(e, carry2):
            alpha = s_blk[e, :] / (dr0[e, :] + dr1[e, :])
            rows[e, :] = rows[e, :] * alpha
            return carry2
        lax.fori_loop(0, CH, row, 0)

        pltpu.sync_copy(rows, acc_sh.at[dst_v], add=True)
        return carry
    lax.fori_loop(0, NCH_W, chunk, 0)
    plsc.subcore_barrier()

    @pl.when(c == 0)
    def _():
        pltpu.sync_copy(acc_sh.at[pl.ds(s * RPS, RPS)],
                        o0_out.at[pl.ds(s * RPS, RPS)])

    @pl.when(c == 1)
    def _():
        pltpu.sync_copy(acc_sh.at[pl.ds(s * RPS, RPS)],
                        o1_out.at[pl.ds(s * RPS, RPS)])


def _final_body(p0_ref, p1_ref, b2_ref, o_ref):
    logits = p0_ref[...] + p1_ref[...] + b2_ref[...]
    m = jnp.max(logits, axis=1, keepdims=True)
    ex = jnp.exp(logits - m)
    lse = jnp.log(jnp.sum(ex, axis=1, keepdims=True))
    o_ref[...] = logits - m - lse


def _final(o0, o1, b2r):
    return pl.pallas_call(
        _final_body,
        grid=(NBLK,),
        in_specs=[
            pl.BlockSpec((BLK, CLS), lambda i: (i, 0)),
            pl.BlockSpec((BLK, CLS), lambda i: (i, 0)),
            pl.BlockSpec((1, CLS), lambda i: (0, 0)),
        ],
        out_specs=pl.BlockSpec((BLK, CLS), lambda i: (i, 0)),
        out_shape=jax.ShapeDtypeStruct((NPAD, CLS), jnp.float32),
    )(o0, o1, b2r)


def kernel(x, edge_index, W1, att_src1, att_dst1, b1, W2, att_src2, att_dst2, b2):
    xp = jnp.concatenate(
        [x.astype(jnp.float32), jnp.zeros((NPAD - N, F), jnp.float32)])
    loop = jnp.arange(N, dtype=jnp.int32)
    pad = EPAD - E
    src = jnp.concatenate([edge_index[0].astype(jnp.int32), loop,
                           jnp.zeros((pad,), jnp.int32)])
    dst = jnp.concatenate([edge_index[1].astype(jnp.int32), loop,
                           jnp.full((pad,), N, jnp.int32)])

    h3, asrc1, adst1 = _mm1(xp, W1, att_src1, att_dst1)
    s1, d10, d11 = _edge_softmax(asrc1, adst1, src, dst)
    al1 = _alpha1(s1, dst, d10, d11)
    out1f = _msg1(h3.reshape(16 * NPAD, 128), src, dst, al1)

    ev = 2 * jnp.arange(16, dtype=jnp.int32)
    perm32 = jnp.concatenate([ev, ev + 1])
    perm128 = jnp.concatenate([b * 32 + perm32 for b in range(4)])
    h2, a2s, a2d = _mm2(out1f.reshape(16, NPAD, 128),
                        W2.reshape(16, 128, CLS)[:, perm128, :],
                        b1.reshape(16, 128)[:, perm128],
                        att_src2, att_dst2)
    s2, d20, d21 = _edge_softmax(a2s, a2d, src, dst)
    o20, o21 = _msg2(h2, src, dst, s2, d20, d21)

    out = _final(o20, o21, b2.reshape(1, CLS))
    return out[:N]
